# Initial kernel scaffold; baseline (speedup 1.0000x reference)
import jax
import jax.numpy as jnp
from jax.experimental import pallas as pl


def kernel(grid_nfeat, mesh_nfeat, edge_index, grid2mesh_efeat, params):
    raise NotImplementedError("write your pallas kernel here")


# speedup vs baseline: 2.8086x; 2.8086x over previous
import functools

import jax
import jax.numpy as jnp
from jax import lax
from jax.experimental import pallas as pl
from jax.experimental.pallas import tpu as pltpu
from jax.experimental.pallas import tpu_sc as plsc

D = 128
N_GRID = 10000
N_MESH = 10000
E = 320000

_NC = 2
_NS = 16
_NW = _NC * _NS
_EPW = E // _NW
_CH = 80
_NCHUNK = _EPW // _CH
_NPAD = 10240
_STRIPE = _NPAD // _NS


def _silu(x):
    return x * jax.nn.sigmoid(x)


def _ln(y, g, bt):
    mu = jnp.mean(y, axis=-1, keepdims=True)
    yc = y - mu
    var = jnp.mean(yc * yc, axis=-1, keepdims=True)
    return yc * lax.rsqrt(var + 1e-5) * g + bt


def _full_spec(a):
    nd = a.ndim
    return pl.BlockSpec(a.shape, lambda i, _n=nd: (0,) * _n)


def _row_spec(rows, cols):
    return pl.BlockSpec((rows, cols), lambda i: (i, 0))


def _grid_body(x, w1, b1, w2, b2, gg, gbt, nw1, nb1, nw2, nb2, ng, nbt, ws,
               gout_ref, gs_ref):
    h = _silu(jnp.dot(x[...], w1[...], preferred_element_type=jnp.float32)
              + b1[...])
    g = _ln(jnp.dot(h, w2[...], preferred_element_type=jnp.float32) + b2[...],
            gg[...], gbt[...])
    h2 = _silu(jnp.dot(g, nw1[...], preferred_element_type=jnp.float32)
               + nb1[...])
    y2 = jnp.dot(h2, nw2[...], preferred_element_type=jnp.float32) + nb2[...]
    gout_ref[...] = g + _ln(y2, ng[...], nbt[...])
    gs_ref[...] = jnp.dot(g, ws[...], preferred_element_type=jnp.float32)


def _mesh_body(x, w1, b1, w2, b2, gg, gbt, wd, m_ref, md_ref):
    h = _silu(jnp.dot(x[...], w1[...], preferred_element_type=jnp.float32)
              + b1[...])
    m = _ln(jnp.dot(h, w2[...], preferred_element_type=jnp.float32) + b2[...],
            gg[...], gbt[...])
    m_ref[...] = m
    md_ref[...] = jnp.dot(m, wd[...], preferred_element_type=jnp.float32)


def _edge_body(ef, srcp, dstp, ew1, eb1, ew2, eb2, eg, ebt,
               we, ib1, iw2, ib2, ig, ibt, eout_ref, enew_ref):
    h0 = _silu(jnp.dot(ef[...], ew1[...], preferred_element_type=jnp.float32)
               + eb1[...])
    e = _ln(jnp.dot(h0, ew2[...], preferred_element_type=jnp.float32)
            + eb2[...], eg[...], ebt[...])
    pre = (jnp.dot(e, we[...], preferred_element_type=jnp.float32)
           + srcp[...] + dstp[...] + ib1[...])
    h = _silu(pre)
    en = _ln(jnp.dot(h, iw2[...], preferred_element_type=jnp.float32)
             + ib2[...], ig[...], ibt[...])
    enew_ref[...] = en
    eout_ref[...] = e + en


def _node_body(m, p0, p1, wa, wm, b1, w2, b2, gg, bt, mout_ref):
    agg = p0[...] + p1[...]
    h = _silu(jnp.dot(agg, wa[...], preferred_element_type=jnp.float32)
              + jnp.dot(m[...], wm[...], preferred_element_type=jnp.float32)
              + b1[...])
    mn = _ln(jnp.dot(h, w2[...], preferred_element_type=jnp.float32) + b2[...],
             gg[...], bt[...])
    mout_ref[...] = m[...] + mn


def _run_rows(body, grid_n, row_block, ins, n_out, n_blocked=1):
    out_shape = tuple(jax.ShapeDtypeStruct((grid_n * row_block, D), jnp.float32)
                      for _ in range(n_out))
    in_specs = [_row_spec(row_block, a.shape[-1]) if k < n_blocked
                else _full_spec(a) for k, a in enumerate(ins)]
    out_specs = tuple(_row_spec(row_block, D) for _ in range(n_out))
    return pl.pallas_call(
        body,
        grid=(grid_n,),
        in_specs=in_specs,
        out_specs=out_specs if n_out > 1 else out_specs[0],
        out_shape=out_shape if n_out > 1 else out_shape[0],
    )(*ins)


@functools.lru_cache(maxsize=None)
def _build_sc_gather():
    mesh = plsc.VectorSubcoreMesh(core_axis_name="c", subcore_axis_name="s")

    @functools.partial(
        pl.kernel, mesh=mesh,
        out_type=(jax.ShapeDtypeStruct((E, D), jnp.float32),
                  jax.ShapeDtypeStruct((E, D), jnp.float32)),
        scratch_types=[pltpu.VMEM((_NCHUNK, _CH), jnp.int32),
                       pltpu.VMEM((_NCHUNK, _CH), jnp.int32),
                       pltpu.VMEM((_CH, D), jnp.float32),
                       pltpu.VMEM((_CH, D), jnp.float32),
                       pltpu.SemaphoreType.DMA,
                       pltpu.SemaphoreType.DMA],
    )
    def sc_gather(gs_hbm, md_hbm, idx0_hbm, idx1_hbm, srcp_hbm, dstp_hbm,
                  idx0_v, idx1_v, rows0_v, rows1_v, sem0, sem1):
        wid = lax.axis_index("s") * _NC + lax.axis_index("c")
        pltpu.sync_copy(idx0_hbm.at[wid], idx0_v)
        pltpu.sync_copy(idx1_hbm.at[wid], idx1_v)

        def body(j, carry):
            base = wid * _EPW + j * _CH
            cp0 = pltpu.async_copy(gs_hbm.at[idx0_v.at[j]], rows0_v, sem0)
            cp1 = pltpu.async_copy(md_hbm.at[idx1_v.at[j]], rows1_v, sem1)
            cp0.wait()
            cp1.wait()
            pltpu.sync_copy(rows0_v, srcp_hbm.at[pl.ds(base, _CH)])
            pltpu.sync_copy(rows1_v, dstp_hbm.at[pl.ds(base, _CH)])
            return carry

        lax.fori_loop(0, _NCHUNK, body, 0)

    return sc_gather


@functools.lru_cache(maxsize=None)
def _build_sc_scatter():
    mesh = plsc.VectorSubcoreMesh(core_axis_name="c", subcore_axis_name="s")

    @functools.partial(
        pl.kernel, mesh=mesh,
        out_type=jax.ShapeDtypeStruct((_NC, _NPAD, D), jnp.float32),
        scratch_types=[pltpu.VMEM((_NCHUNK, _CH), jnp.int32),
                       pltpu.VMEM((_CH, D), jnp.float32),
                       pltpu.VMEM_SHARED((_NPAD, D), jnp.float32)],
    )
    def sc_scatter(enew_hbm, idx1_hbm, zeros_hbm, out_hbm,
                   idx_v, rows_v, acc_sh):
        cid = lax.axis_index("c")
        sid = lax.axis_index("s")
        wid = sid * _NC + cid
        pltpu.sync_copy(zeros_hbm, acc_sh.at[pl.ds(sid * _STRIPE, _STRIPE)])
        plsc.subcore_barrier()
        pltpu.sync_copy(idx1_hbm.at[wid], idx_v)

        def body(j, carry):
            base = wid * _EPW + j * _CH
            pltpu.sync_copy(enew_hbm.at[pl.ds(base, _CH)], rows_v)
            pltpu.sync_copy(rows_v, acc_sh.at[idx_v.at[j]], add=True)
            return carry

        lax.fori_loop(0, _NCHUNK, body, 0)
        plsc.subcore_barrier()
        pltpu.sync_copy(acc_sh.at[pl.ds(sid * _STRIPE, _STRIPE)],
                        out_hbm.at[cid, pl.ds(sid * _STRIPE, _STRIPE)])

    return sc_scatter


def _sc_gather(gs, md, idx0, idx1):
    return _build_sc_gather()(gs, md, idx0, idx1)


def _sc_scatter(e_new, idx1, zeros):
    return _build_sc_scatter()(e_new, idx1, zeros)


def kernel(grid_nfeat, mesh_nfeat, edge_index, grid2mesh_efeat, params):
    p = params

    def vec(w, name):
        return w[name].reshape(1, -1)

    in_w1 = p["in_edge"]["W1"]
    w1e, w1s, w1d = in_w1[0:D], in_w1[D:2 * D], in_w1[2 * D:3 * D]
    in_node_w1 = p["in_node"]["W1"]
    wa, wm = in_node_w1[0:D], in_node_w1[D:2 * D]

    ge = p["grid_emb"]
    gn = p["grid_node"]
    g_out, gs = _run_rows(
        _grid_body, 25, 400,
        [grid_nfeat, ge["W1"], vec(ge, "b1"), ge["W2"], vec(ge, "b2"),
         vec(ge, "g"), vec(ge, "bt"),
         gn["W1"], vec(gn, "b1"), gn["W2"], vec(gn, "b2"),
         vec(gn, "g"), vec(gn, "bt"), w1s], 2)

    me = p["mesh_emb"]
    m, md = _run_rows(
        _mesh_body, 25, 400,
        [mesh_nfeat, me["W1"], vec(me, "b1"), me["W2"], vec(me, "b2"),
         vec(me, "g"), vec(me, "bt"), w1d], 2)

    idx0 = edge_index[0].reshape(_NW, _NCHUNK, _CH)
    idx1 = edge_index[1].reshape(_NW, _NCHUNK, _CH)
    srcp, dstp = _sc_gather(gs, md, idx0, idx1)

    ee = p["edge_emb"]
    ie = p["in_edge"]
    e_out, e_new = _run_rows(
        _edge_body, 625, 512,
        [grid2mesh_efeat, srcp, dstp,
         ee["W1"], vec(ee, "b1"), ee["W2"], vec(ee, "b2"),
         vec(ee, "g"), vec(ee, "bt"),
         w1e, vec(ie, "b1"), ie["W2"], vec(ie, "b2"),
         vec(ie, "g"), vec(ie, "bt")], 2, n_blocked=3)

    zeros = jnp.zeros((_STRIPE, D), jnp.float32)
    partials = _sc_scatter(e_new, idx1, zeros)
    p0 = partials[0, :N_MESH]
    p1 = partials[1, :N_MESH]

    inn = p["in_node"]
    m_out = _run_rows(
        _node_body, 25, 400,
        [m, p0, p1, wa, wm, vec(inn, "b1"), inn["W2"], vec(inn, "b2"),
         vec(inn, "g"), vec(inn, "bt")], 1, n_blocked=3)

    return (g_out, m_out, e_out)


# speedup vs baseline: 4.0339x; 1.4363x over previous
import functools

import jax
import jax.numpy as jnp
from jax import lax
from jax.experimental import pallas as pl
from jax.experimental.pallas import tpu as pltpu
from jax.experimental.pallas import tpu_sc as plsc

D = 128
N_GRID = 10000
N_MESH = 10000
E = 320000

_NC = 2
_NS = 16
_NW = _NC * _NS
_EPW = E // _NW
_CH = 80
_NCHUNK = _EPW // _CH
_NPAD = 10240
_STRIPE = _NPAD // _NS


def _silu(x):
    return x * jax.nn.sigmoid(x)


def _ln(y, g, bt):
    mu = jnp.mean(y, axis=-1, keepdims=True)
    yc = y - mu
    var = jnp.mean(yc * yc, axis=-1, keepdims=True)
    return yc * lax.rsqrt(var + 1e-5) * g + bt


def _full_spec(a):
    nd = a.ndim
    return pl.BlockSpec(a.shape, lambda i, _n=nd: (0,) * _n)


def _row_spec(rows, cols):
    return pl.BlockSpec((rows, cols), lambda i: (i, 0))


def _grid_body(x, w1, b1, w2, b2, gg, gbt, nw1, nb1, nw2, nb2, ng, nbt, ws,
               gout_ref, gs_ref):
    h = _silu(jnp.dot(x[...], w1[...], preferred_element_type=jnp.float32)
              + b1[...])
    g = _ln(jnp.dot(h, w2[...], preferred_element_type=jnp.float32) + b2[...],
            gg[...], gbt[...])
    h2 = _silu(jnp.dot(g, nw1[...], preferred_element_type=jnp.float32)
               + nb1[...])
    y2 = jnp.dot(h2, nw2[...], preferred_element_type=jnp.float32) + nb2[...]
    gout_ref[...] = g + _ln(y2, ng[...], nbt[...])
    gs_ref[...] = jnp.dot(g, ws[...], preferred_element_type=jnp.float32)


def _mesh_body(x, w1, b1, w2, b2, gg, gbt, wd, m_ref, md_ref):
    h = _silu(jnp.dot(x[...], w1[...], preferred_element_type=jnp.float32)
              + b1[...])
    m = _ln(jnp.dot(h, w2[...], preferred_element_type=jnp.float32) + b2[...],
            gg[...], gbt[...])
    m_ref[...] = m
    md_ref[...] = jnp.dot(m, wd[...], preferred_element_type=jnp.float32)


def _edge_body(ef, sump, ew1, eb1, ew2, eb2, eg, ebt,
               we, ib1, iw2, ib2, ig, ibt, eout_ref, enew_ref):
    h0 = _silu(jnp.dot(ef[...], ew1[...], preferred_element_type=jnp.float32)
               + eb1[...])
    e = _ln(jnp.dot(h0, ew2[...], preferred_element_type=jnp.float32)
            + eb2[...], eg[...], ebt[...])
    pre = (jnp.dot(e, we[...], preferred_element_type=jnp.float32) + ib1[...]
           + sump[...])
    h = _silu(pre)
    en = _ln(jnp.dot(h, iw2[...], preferred_element_type=jnp.float32)
             + ib2[...], ig[...], ibt[...])
    enew_ref[...] = en
    eout_ref[...] = e + en


def _node_body(m, p0, p1, wa, wm, b1, w2, b2, gg, bt, mout_ref):
    agg = p0[...] + p1[...]
    h = _silu(jnp.dot(agg, wa[...], preferred_element_type=jnp.float32)
              + jnp.dot(m[...], wm[...], preferred_element_type=jnp.float32)
              + b1[...])
    mn = _ln(jnp.dot(h, w2[...], preferred_element_type=jnp.float32) + b2[...],
             gg[...], bt[...])
    mout_ref[...] = m[...] + mn


def _run_rows(body, grid_n, row_block, ins, outs, n_blocked=1):
    out_shape = tuple(jax.ShapeDtypeStruct((grid_n * row_block, c), dt)
                      for c, dt in outs)
    in_specs = [_row_spec(row_block, a.shape[-1]) if k < n_blocked
                else _full_spec(a) for k, a in enumerate(ins)]
    out_specs = tuple(_row_spec(row_block, c) for c, _ in outs)
    one = len(outs) == 1
    return pl.pallas_call(
        body,
        grid=(grid_n,),
        in_specs=in_specs,
        out_specs=out_specs[0] if one else out_specs,
        out_shape=out_shape[0] if one else out_shape,
    )(*ins)


@functools.lru_cache(maxsize=None)
def _build_sc_gather():
    mesh = plsc.VectorSubcoreMesh(core_axis_name="c", subcore_axis_name="s")

    @functools.partial(
        pl.kernel, mesh=mesh,
        out_type=jax.ShapeDtypeStruct((E, D), jnp.float32),
        scratch_types=[pltpu.VMEM((_NCHUNK, _CH), jnp.int32),
                       pltpu.VMEM((_NCHUNK, _CH), jnp.int32),
                       pltpu.VMEM((_CH, D), jnp.float32),
                       pltpu.VMEM((_CH, D), jnp.float32),
                       pltpu.VMEM((_CH, D), jnp.float32),
                       pltpu.VMEM((_CH, D), jnp.float32),
                       pltpu.SemaphoreType.DMA,
                       pltpu.SemaphoreType.DMA,
                       pltpu.SemaphoreType.DMA,
                       pltpu.SemaphoreType.DMA,
                       pltpu.SemaphoreType.DMA],
    )
    def sc_gather(gs_hbm, md_hbm, idx0_hbm, idx1_hbm, sum_hbm,
                  idx0_v, idx1_v, r0a, r1a, r0b, r1b,
                  sg0, sg1, sg2, sg3, sw):
        wid = lax.axis_index("s") * _NC + lax.axis_index("c")
        pltpu.sync_copy(idx0_hbm.at[wid], idx0_v)
        pltpu.sync_copy(idx1_hbm.at[wid], idx1_v)

        def add_into(dst, src):
            def add_body(i, carry):
                for q in range(D // 16):
                    o = q * 16
                    dst[i, pl.ds(o, 16)] = (dst[i, pl.ds(o, 16)]
                                            + src[i, pl.ds(o, 16)])
                return carry
            lax.fori_loop(0, _CH, add_body, 0)

        def start(c, ra, rb, sa, sb):
            ga = pltpu.async_copy(gs_hbm.at[idx0_v.at[c]], ra, sa)
            gb = pltpu.async_copy(md_hbm.at[idx1_v.at[c]], rb, sb)
            return ga, gb

        def body(k, carry):
            c0 = 2 * k
            c1 = c0 + 1
            b0 = wid * _EPW + c0 * _CH
            b1 = wid * _EPW + c1 * _CH
            g0, g1 = start(c0, r0a, r1a, sg0, sg1)
            g2, g3 = start(c1, r0b, r1b, sg2, sg3)
            g0.wait()
            g1.wait()
            add_into(r0a, r1a)
            w0 = pltpu.async_copy(r0a, sum_hbm.at[pl.ds(b0, _CH)], sw)
            g2.wait()
            g3.wait()
            add_into(r0b, r1b)
            w1 = pltpu.async_copy(r0b, sum_hbm.at[pl.ds(b1, _CH)], sw)
            w0.wait()
            w1.wait()
            return carry

        lax.fori_loop(0, _NCHUNK // 2, body, 0)
        ct = _NCHUNK - 1
        bt = wid * _EPW + ct * _CH
        g0, g1 = start(ct, r0a, r1a, sg0, sg1)
        g0.wait()
        g1.wait()
        add_into(r0a, r1a)
        pltpu.sync_copy(r0a, sum_hbm.at[pl.ds(bt, _CH)])

    return sc_gather


@functools.lru_cache(maxsize=None)
def _build_sc_scatter():
    mesh = plsc.VectorSubcoreMesh(core_axis_name="c", subcore_axis_name="s")

    @functools.partial(
        pl.kernel, mesh=mesh,
        out_type=jax.ShapeDtypeStruct((_NC, _NPAD, D), jnp.float32),
        scratch_types=[pltpu.VMEM((_NCHUNK, _CH), jnp.int32),
                       pltpu.VMEM((_CH, D), jnp.float32),
                       pltpu.VMEM((_CH, D), jnp.float32),
                       pltpu.VMEM_SHARED((_NPAD, D), jnp.float32),
                       pltpu.SemaphoreType.DMA,
                       pltpu.SemaphoreType.DMA,
                       pltpu.SemaphoreType.DMA,
                       pltpu.SemaphoreType.DMA],
    )
    def sc_scatter(enew_hbm, idx1_hbm, zeros_hbm, out_hbm,
                   idx_v, ra, rb, acc_sh, sr0, sr1, sa0, sa1):
        cid = lax.axis_index("c")
        sid = lax.axis_index("s")
        wid = sid * _NC + cid
        pltpu.sync_copy(zeros_hbm, acc_sh.at[pl.ds(sid * _STRIPE, _STRIPE)])
        plsc.subcore_barrier()
        pltpu.sync_copy(idx1_hbm.at[wid], idx_v)

        def body(k, carry):
            c0 = 2 * k
            c1 = c0 + 1
            b0 = wid * _EPW + c0 * _CH
            b1 = wid * _EPW + c1 * _CH
            r0 = pltpu.async_copy(enew_hbm.at[pl.ds(b0, _CH)], ra, sr0)
            r1 = pltpu.async_copy(enew_hbm.at[pl.ds(b1, _CH)], rb, sr1)
            r0.wait()
            a0 = pltpu.async_copy(ra, acc_sh.at[idx_v.at[c0]], sa0, add=True)
            r1.wait()
            a1 = pltpu.async_copy(rb, acc_sh.at[idx_v.at[c1]], sa1, add=True)
            a0.wait()
            a1.wait()
            return carry

        lax.fori_loop(0, _NCHUNK // 2, body, 0)
        ct = _NCHUNK - 1
        bt = wid * _EPW + ct * _CH
        pltpu.sync_copy(enew_hbm.at[pl.ds(bt, _CH)], ra)
        pltpu.sync_copy(ra, acc_sh.at[idx_v.at[ct]], add=True)
        plsc.subcore_barrier()
        pltpu.sync_copy(acc_sh.at[pl.ds(sid * _STRIPE, _STRIPE)],
                        out_hbm.at[cid, pl.ds(sid * _STRIPE, _STRIPE)])

    return sc_scatter


def _sc_gather(gs, md, idx0, idx1):
    return _build_sc_gather()(gs, md, idx0, idx1)


def _sc_scatter(e_new, idx1, zeros):
    return _build_sc_scatter()(e_new, idx1, zeros)


def kernel(grid_nfeat, mesh_nfeat, edge_index, grid2mesh_efeat, params):
    p = params

    def vec(w, name):
        return w[name].reshape(1, -1)

    in_w1 = p["in_edge"]["W1"]
    w1e, w1s, w1d = in_w1[0:D], in_w1[D:2 * D], in_w1[2 * D:3 * D]
    in_node_w1 = p["in_node"]["W1"]
    wa, wm = in_node_w1[0:D], in_node_w1[D:2 * D]

    ge = p["grid_emb"]
    gn = p["grid_node"]
    g_out, gs = _run_rows(
        _grid_body, 25, 400,
        [grid_nfeat, ge["W1"], vec(ge, "b1"), ge["W2"], vec(ge, "b2"),
         vec(ge, "g"), vec(ge, "bt"),
         gn["W1"], vec(gn, "b1"), gn["W2"], vec(gn, "b2"),
         vec(gn, "g"), vec(gn, "bt"), w1s],
        [(D, jnp.float32), (D, jnp.float32)])

    me = p["mesh_emb"]
    m, md = _run_rows(
        _mesh_body, 25, 400,
        [mesh_nfeat, me["W1"], vec(me, "b1"), me["W2"], vec(me, "b2"),
         vec(me, "g"), vec(me, "bt"), w1d],
        [(D, jnp.float32), (D, jnp.float32)])

    idx0 = edge_index[0].reshape(_NW, _NCHUNK, _CH)
    idx1 = edge_index[1].reshape(_NW, _NCHUNK, _CH)
    sump = _sc_gather(gs, md, idx0, idx1)

    ee = p["edge_emb"]
    ie = p["in_edge"]
    e_out, e_new = _run_rows(
        _edge_body, 250, 1280,
        [grid2mesh_efeat, sump,
         ee["W1"], vec(ee, "b1"), ee["W2"], vec(ee, "b2"),
         vec(ee, "g"), vec(ee, "bt"),
         w1e, vec(ie, "b1"), ie["W2"], vec(ie, "b2"),
         vec(ie, "g"), vec(ie, "bt")],
        [(D, jnp.float32), (D, jnp.float32)], n_blocked=2)

    zeros = jnp.zeros((_STRIPE, D), jnp.float32)
    partials = _sc_scatter(e_new, idx1, zeros)
    p0 = partials[0, :N_MESH]
    p1 = partials[1, :N_MESH]

    inn = p["in_node"]
    m_out = _run_rows(
        _node_body, 25, 400,
        [m, p0, p1, wa, wm, vec(inn, "b1"), inn["W2"], vec(inn, "b2"),
         vec(inn, "g"), vec(inn, "bt")],
        [(D, jnp.float32)], n_blocked=3)

    return (g_out, m_out, e_out)


# speedup vs baseline: 4.2920x; 1.0640x over previous
import functools

import jax
import jax.numpy as jnp
from jax import lax
from jax.experimental import pallas as pl
from jax.experimental.pallas import tpu as pltpu
from jax.experimental.pallas import tpu_sc as plsc

D = 128
N_GRID = 10000
N_MESH = 10000
E = 320000

_NC = 2
_NS = 16
_NW = _NC * _NS
_EPW = E // _NW
_CH = 80
_NCHUNK = _EPW // _CH
_NPAD = 10240
_STRIPE = _NPAD // _NS


def _silu(x):
    return x * jax.nn.sigmoid(x)


def _ln(y, g, bt):
    mu = jnp.mean(y, axis=-1, keepdims=True)
    yc = y - mu
    var = jnp.mean(yc * yc, axis=-1, keepdims=True)
    return yc * lax.rsqrt(var + 1e-5) * g + bt


def _full_spec(a):
    nd = a.ndim
    return pl.BlockSpec(a.shape, lambda i, _n=nd: (0,) * _n)


def _row_spec(rows, cols):
    return pl.BlockSpec((rows, cols), lambda i: (i, 0))


def _gm_body(xg, xm,
             gw1, gb1, gw2, gb2, gg, gbt,
             nw1, nb1, nw2, nb2, ng, nbt, ws,
             mw1, mb1, mw2, mb2, mg, mbt, wd,
             gout_ref, gs_ref, m_ref, md_ref):
    h = _silu(jnp.dot(xg[...], gw1[...], preferred_element_type=jnp.float32)
              + gb1[...])
    g = _ln(jnp.dot(h, gw2[...], preferred_element_type=jnp.float32)
            + gb2[...], gg[...], gbt[...])
    h2 = _silu(jnp.dot(g, nw1[...], preferred_element_type=jnp.float32)
               + nb1[...])
    y2 = jnp.dot(h2, nw2[...], preferred_element_type=jnp.float32) + nb2[...]
    gout_ref[...] = g + _ln(y2, ng[...], nbt[...])
    gs_ref[...] = jnp.dot(g, ws[...], preferred_element_type=jnp.float32)
    hm = _silu(jnp.dot(xm[...], mw1[...], preferred_element_type=jnp.float32)
               + mb1[...])
    m = _ln(jnp.dot(hm, mw2[...], preferred_element_type=jnp.float32)
            + mb2[...], mg[...], mbt[...])
    m_ref[...] = m
    md_ref[...] = jnp.dot(m, wd[...], preferred_element_type=jnp.float32)


def _edge_body(ef, sump, ew1, eb1, ew2, eb2, eg, ebt,
               we, ib1, iw2, ib2, ig, ibt, eout_ref, enew_ref):
    h0 = _silu(jnp.dot(ef[...], ew1[...], preferred_element_type=jnp.float32)
               + eb1[...])
    e = _ln(jnp.dot(h0, ew2[...], preferred_element_type=jnp.float32)
            + eb2[...], eg[...], ebt[...])
    pre = (jnp.dot(e, we[...], preferred_element_type=jnp.float32) + ib1[...]
           + sump[...])
    h = _silu(pre)
    en = _ln(jnp.dot(h, iw2[...], preferred_element_type=jnp.float32)
             + ib2[...], ig[...], ibt[...])
    enew_ref[...] = en
    eout_ref[...] = e + en


def _node_body(m, p0, p1, wa, wm, b1, w2, b2, gg, bt, mout_ref):
    agg = p0[...] + p1[...]
    h = _silu(jnp.dot(agg, wa[...], preferred_element_type=jnp.float32)
              + jnp.dot(m[...], wm[...], preferred_element_type=jnp.float32)
              + b1[...])
    mn = _ln(jnp.dot(h, w2[...], preferred_element_type=jnp.float32) + b2[...],
             gg[...], bt[...])
    mout_ref[...] = m[...] + mn


def _run_rows(body, grid_n, row_block, ins, outs, n_blocked=1):
    out_shape = tuple(jax.ShapeDtypeStruct((grid_n * row_block, c), dt)
                      for c, dt in outs)
    in_specs = [_row_spec(row_block, a.shape[-1]) if k < n_blocked
                else _full_spec(a) for k, a in enumerate(ins)]
    out_specs = tuple(_row_spec(row_block, c) for c, _ in outs)
    one = len(outs) == 1
    return pl.pallas_call(
        body,
        grid=(grid_n,),
        in_specs=in_specs,
        out_specs=out_specs[0] if one else out_specs,
        out_shape=out_shape[0] if one else out_shape,
    )(*ins)


@functools.lru_cache(maxsize=None)
def _build_sc_gather():
    mesh = plsc.VectorSubcoreMesh(core_axis_name="c", subcore_axis_name="s")
    P = 4
    NRING = (_NCHUNK // P) * P

    @functools.partial(
        pl.kernel, mesh=mesh,
        out_type=jax.ShapeDtypeStruct((E, D), jnp.float32),
        scratch_types=[pltpu.VMEM((_NCHUNK, _CH), jnp.int32),
                       pltpu.VMEM((_NCHUNK, _CH), jnp.int32)]
                      + [pltpu.VMEM((_CH, D), jnp.float32)] * (2 * P)
                      + [pltpu.SemaphoreType.DMA] * (3 * P),
    )
    def sc_gather(gs_hbm, md_hbm, idx0_hbm, idx1_hbm, sum_hbm,
                  idx0_v, idx1_v, *bufsems):
        ra = bufsems[0:P]
        rb = bufsems[P:2 * P]
        sga = bufsems[2 * P:3 * P]
        sgb = bufsems[3 * P:4 * P]
        sw = bufsems[4 * P:5 * P]
        wid = lax.axis_index("s") * _NC + lax.axis_index("c")
        pltpu.sync_copy(idx0_hbm.at[wid], idx0_v)
        pltpu.sync_copy(idx1_hbm.at[wid], idx1_v)

        def add_into(dst, src):
            def add_body(i, carry):
                for q in range(D // 16):
                    o = q * 16
                    dst[i, pl.ds(o, 16)] = (dst[i, pl.ds(o, 16)]
                                            + src[i, pl.ds(o, 16)])
                return carry
            lax.fori_loop(0, _CH, add_body, 0)

        def start(c, u):
            pltpu.async_copy(gs_hbm.at[idx0_v.at[c]], ra[u], sga[u])
            pltpu.async_copy(md_hbm.at[idx1_v.at[c]], rb[u], sgb[u])

        def wait_gather(c, u):
            pltpu.make_async_copy(gs_hbm.at[idx0_v.at[c]], ra[u],
                                  sga[u]).wait()
            pltpu.make_async_copy(md_hbm.at[idx1_v.at[c]], rb[u],
                                  sgb[u]).wait()

        for u in range(P):
            start(u, u)

        def body(k, carry):
            for u in range(P):
                c = P * k + u
                b = wid * _EPW + c * _CH
                wait_gather(c, u)
                add_into(ra[u], rb[u])
                pltpu.async_copy(ra[u], sum_hbm.at[pl.ds(b, _CH)], sw[u])
            for u in range(P):
                c = P * k + u
                cn = c + P
                b = wid * _EPW + c * _CH
                pltpu.make_async_copy(ra[u], sum_hbm.at[pl.ds(b, _CH)],
                                      sw[u]).wait()

                @pl.when(cn < NRING)
                def _():
                    start(cn, u)
            return carry

        lax.fori_loop(0, NRING // P, body, 0)
        for ct in range(NRING, _NCHUNK):
            bt = wid * _EPW + ct * _CH
            start(ct, 0)
            wait_gather(ct, 0)
            add_into(ra[0], rb[0])
            pltpu.sync_copy(ra[0], sum_hbm.at[pl.ds(bt, _CH)])

    return sc_gather


@functools.lru_cache(maxsize=None)
def _build_sc_scatter():
    mesh = plsc.VectorSubcoreMesh(core_axis_name="c", subcore_axis_name="s")

    P = 2
    NRING = (_NCHUNK // P) * P

    @functools.partial(
        pl.kernel, mesh=mesh,
        out_type=jax.ShapeDtypeStruct((_NC, _NPAD, D), jnp.float32),
        scratch_types=[pltpu.VMEM((_NCHUNK, _CH), jnp.int32)]
                      + [pltpu.VMEM((_CH, D), jnp.float32)] * P
                      + [pltpu.VMEM_SHARED((_NPAD, D), jnp.float32)]
                      + [pltpu.SemaphoreType.DMA] * (2 * P),
    )
    def sc_scatter(enew_hbm, idx1_hbm, zeros_hbm, out_hbm, idx_v, *rest):
        rbuf = rest[0:P]
        acc_sh = rest[P]
        sr = rest[P + 1:2 * P + 1]
        sa = rest[2 * P + 1:3 * P + 1]
        cid = lax.axis_index("c")
        sid = lax.axis_index("s")
        wid = sid * _NC + cid
        pltpu.sync_copy(zeros_hbm, acc_sh.at[pl.ds(sid * _STRIPE, _STRIPE)])
        plsc.subcore_barrier()
        pltpu.sync_copy(idx1_hbm.at[wid], idx_v)

        def start_read(c, u):
            b = wid * _EPW + c * _CH
            pltpu.async_copy(enew_hbm.at[pl.ds(b, _CH)], rbuf[u], sr[u])

        def wait_read(c, u):
            b = wid * _EPW + c * _CH
            pltpu.make_async_copy(enew_hbm.at[pl.ds(b, _CH)], rbuf[u],
                                  sr[u]).wait()

        for u in range(P):
            start_read(u, u)

        def body(k, carry):
            for u in range(P):
                c = P * k + u
                wait_read(c, u)
                pltpu.async_copy(rbuf[u], acc_sh.at[idx_v.at[c]], sa[u],
                                 add=True)
            for u in range(P):
                c = P * k + u
                cn = c + P
                pltpu.make_async_copy(rbuf[u], acc_sh.at[idx_v.at[c]],
                                      sa[u]).wait()

                @pl.when(cn < NRING)
                def _():
                    start_read(cn, u)
            return carry

        lax.fori_loop(0, NRING // P, body, 0)
        for ct in range(NRING, _NCHUNK):
            start_read(ct, 0)
            wait_read(ct, 0)
            pltpu.sync_copy(rbuf[0], acc_sh.at[idx_v.at[ct]], add=True)
        plsc.subcore_barrier()
        pltpu.sync_copy(acc_sh.at[pl.ds(sid * _STRIPE, _STRIPE)],
                        out_hbm.at[cid, pl.ds(sid * _STRIPE, _STRIPE)])

    return sc_scatter


def _sc_gather(gs, md, idx0, idx1):
    return _build_sc_gather()(gs, md, idx0, idx1)


def _sc_scatter(e_new, idx1, zeros):
    return _build_sc_scatter()(e_new, idx1, zeros)


def kernel(grid_nfeat, mesh_nfeat, edge_index, grid2mesh_efeat, params):
    p = params

    def vec(w, name):
        return w[name].reshape(1, -1)

    in_w1 = p["in_edge"]["W1"]
    w1e, w1s, w1d = in_w1[0:D], in_w1[D:2 * D], in_w1[2 * D:3 * D]
    in_node_w1 = p["in_node"]["W1"]
    wa, wm = in_node_w1[0:D], in_node_w1[D:2 * D]

    ge = p["grid_emb"]
    gn = p["grid_node"]
    me = p["mesh_emb"]
    g_out, gs, m, md = _run_rows(
        _gm_body, 25, 400,
        [grid_nfeat, mesh_nfeat,
         ge["W1"], vec(ge, "b1"), ge["W2"], vec(ge, "b2"),
         vec(ge, "g"), vec(ge, "bt"),
         gn["W1"], vec(gn, "b1"), gn["W2"], vec(gn, "b2"),
         vec(gn, "g"), vec(gn, "bt"), w1s,
         me["W1"], vec(me, "b1"), me["W2"], vec(me, "b2"),
         vec(me, "g"), vec(me, "bt"), w1d],
        [(D, jnp.float32)] * 4, n_blocked=2)

    idx0 = edge_index[0].reshape(_NW, _NCHUNK, _CH)
    idx1 = edge_index[1].reshape(_NW, _NCHUNK, _CH)
    sump = _sc_gather(gs, md, idx0, idx1)

    ee = p["edge_emb"]
    ie = p["in_edge"]
    e_out, e_new = _run_rows(
        _edge_body, 250, 1280,
        [grid2mesh_efeat, sump,
         ee["W1"], vec(ee, "b1"), ee["W2"], vec(ee, "b2"),
         vec(ee, "g"), vec(ee, "bt"),
         w1e, vec(ie, "b1"), ie["W2"], vec(ie, "b2"),
         vec(ie, "g"), vec(ie, "bt")],
        [(D, jnp.float32), (D, jnp.float32)], n_blocked=2)

    zeros = jnp.zeros((_STRIPE, D), jnp.float32)
    partials = _sc_scatter(e_new, idx1, zeros)
    p0 = partials[0, :N_MESH]
    p1 = partials[1, :N_MESH]

    inn = p["in_node"]
    m_out = _run_rows(
        _node_body, 25, 400,
        [m, p0, p1, wa, wm, vec(inn, "b1"), inn["W2"], vec(inn, "b2"),
         vec(inn, "g"), vec(inn, "bt")],
        [(D, jnp.float32)], n_blocked=3)

    return (g_out, m_out, e_out)


# speedup vs baseline: 4.6783x; 1.0900x over previous
import functools

import jax
import jax.numpy as jnp
from jax import lax
from jax.experimental import pallas as pl
from jax.experimental.pallas import tpu as pltpu
from jax.experimental.pallas import tpu_sc as plsc

D = 128
N_GRID = 10000
N_MESH = 10000
E = 320000

_NC = 2
_NS = 16
_NW = _NC * _NS
_EPW = E // _NW
_CH = 80
_NCHUNK = _EPW // _CH
_NPAD = 10240
_STRIPE = _NPAD // _NS


def _silu(x):
    return x * jax.nn.sigmoid(x)


def _ln(y, g, bt):
    mu = jnp.mean(y, axis=-1, keepdims=True)
    yc = y - mu
    var = jnp.mean(yc * yc, axis=-1, keepdims=True)
    return yc * lax.rsqrt(var + 1e-5) * g + bt


def _full_spec(a):
    nd = a.ndim
    return pl.BlockSpec(a.shape, lambda i, _n=nd: (0,) * _n)


def _row_spec(rows, cols):
    return pl.BlockSpec((rows, cols), lambda i: (i, 0))


def _gm_body(xg, xm,
             gw1, gb1, gw2, gb2, gg, gbt,
             nw1, nb1, nw2, nb2, ng, nbt, ws,
             mw1, mb1, mw2, mb2, mg, mbt, wd,
             gout_ref, gs_ref, m_ref, md_ref):
    h = _silu(jnp.dot(xg[...], gw1[...], preferred_element_type=jnp.float32)
              + gb1[...])
    g = _ln(jnp.dot(h, gw2[...], preferred_element_type=jnp.float32)
            + gb2[...], gg[...], gbt[...])
    h2 = _silu(jnp.dot(g, nw1[...], preferred_element_type=jnp.float32)
               + nb1[...])
    y2 = jnp.dot(h2, nw2[...], preferred_element_type=jnp.float32) + nb2[...]
    gout_ref[...] = g + _ln(y2, ng[...], nbt[...])
    gs_ref[...] = jnp.dot(g, ws[...], preferred_element_type=jnp.float32)
    hm = _silu(jnp.dot(xm[...], mw1[...], preferred_element_type=jnp.float32)
               + mb1[...])
    m = _ln(jnp.dot(hm, mw2[...], preferred_element_type=jnp.float32)
            + mb2[...], mg[...], mbt[...])
    m_ref[...] = m
    md_ref[...] = jnp.dot(m, wd[...], preferred_element_type=jnp.float32)


def _edge_body(ef, sump, ew1, eb1, ew2, eb2, eg, ebt,
               we, ib1, iw2, ib2, ig, ibt, eout_ref, enew_ref):
    h0 = _silu(jnp.dot(ef[...], ew1[...], preferred_element_type=jnp.float32)
               + eb1[...])
    e = _ln(jnp.dot(h0, ew2[...], preferred_element_type=jnp.float32)
            + eb2[...], eg[...], ebt[...])
    pre = (jnp.dot(e, we[...], preferred_element_type=jnp.float32) + ib1[...]
           + sump[...])
    h = _silu(pre)
    en = _ln(jnp.dot(h, iw2[...], preferred_element_type=jnp.float32)
             + ib2[...], ig[...], ibt[...])
    enew_ref[...] = en
    eout_ref[...] = e + en


def _node_body(m, p0, p1, wa, wm, b1, w2, b2, gg, bt, mout_ref):
    agg = p0[...] + p1[...]
    h = _silu(jnp.dot(agg, wa[...], preferred_element_type=jnp.float32)
              + jnp.dot(m[...], wm[...], preferred_element_type=jnp.float32)
              + b1[...])
    mn = _ln(jnp.dot(h, w2[...], preferred_element_type=jnp.float32) + b2[...],
             gg[...], bt[...])
    mout_ref[...] = m[...] + mn


def _run_rows(body, grid_n, row_block, ins, outs, n_blocked=1):
    out_shape = tuple(jax.ShapeDtypeStruct((grid_n * row_block, c), dt)
                      for c, dt in outs)
    in_specs = [_row_spec(row_block, a.shape[-1]) if k < n_blocked
                else _full_spec(a) for k, a in enumerate(ins)]
    out_specs = tuple(_row_spec(row_block, c) for c, _ in outs)
    one = len(outs) == 1
    return pl.pallas_call(
        body,
        grid=(grid_n,),
        in_specs=in_specs,
        out_specs=out_specs[0] if one else out_specs,
        out_shape=out_shape[0] if one else out_shape,
    )(*ins)


@functools.lru_cache(maxsize=None)
def _build_sc_gather():
    mesh = plsc.VectorSubcoreMesh(core_axis_name="c", subcore_axis_name="s")
    P = 4
    NRING = (_NCHUNK // P) * P

    @functools.partial(
        pl.kernel, mesh=mesh,
        out_type=jax.ShapeDtypeStruct((E, D), jnp.float32),
        scratch_types=[pltpu.VMEM((_NCHUNK, _CH), jnp.int32),
                       pltpu.VMEM((_NCHUNK, _CH), jnp.int32)]
                      + [pltpu.VMEM((_CH, D), jnp.float32)] * (2 * P)
                      + [pltpu.SemaphoreType.DMA] * (3 * P),
    )
    def sc_gather(gs_hbm, md_hbm, idx0_hbm, idx1_hbm, sum_hbm,
                  idx0_v, idx1_v, *bufsems):
        ra = bufsems[0:P]
        rb = bufsems[P:2 * P]
        sga = bufsems[2 * P:3 * P]
        sgb = bufsems[3 * P:4 * P]
        sw = bufsems[4 * P:5 * P]
        wid = lax.axis_index("s") * _NC + lax.axis_index("c")
        pltpu.sync_copy(idx0_hbm.at[wid], idx0_v)
        pltpu.sync_copy(idx1_hbm.at[wid], idx1_v)

        def add_into(dst, src):
            def add_body(i, carry):
                for q in range(D // 16):
                    o = q * 16
                    dst[i, pl.ds(o, 16)] = (dst[i, pl.ds(o, 16)]
                                            + src[i, pl.ds(o, 16)])
                return carry
            lax.fori_loop(0, _CH, add_body, 0)

        def start(c, u):
            pltpu.async_copy(gs_hbm.at[idx0_v.at[c]], ra[u], sga[u])
            pltpu.async_copy(md_hbm.at[idx1_v.at[c]], rb[u], sgb[u])

        def wait_gather(c, u):
            pltpu.make_async_copy(gs_hbm.at[idx0_v.at[c]], ra[u],
                                  sga[u]).wait()
            pltpu.make_async_copy(md_hbm.at[idx1_v.at[c]], rb[u],
                                  sgb[u]).wait()

        for u in range(P):
            start(u, u)

        def body(k, carry):
            for u in range(P):
                c = P * k + u
                b = wid * _EPW + c * _CH
                wait_gather(c, u)
                add_into(ra[u], rb[u])
                pltpu.async_copy(ra[u], sum_hbm.at[pl.ds(b, _CH)], sw[u])
            for u in range(P):
                c = P * k + u
                cn = c + P
                b = wid * _EPW + c * _CH
                pltpu.make_async_copy(ra[u], sum_hbm.at[pl.ds(b, _CH)],
                                      sw[u]).wait()

                @pl.when(cn < NRING)
                def _():
                    start(cn, u)
            return carry

        lax.fori_loop(0, NRING // P, body, 0)
        for ct in range(NRING, _NCHUNK):
            bt = wid * _EPW + ct * _CH
            start(ct, 0)
            wait_gather(ct, 0)
            add_into(ra[0], rb[0])
            pltpu.sync_copy(ra[0], sum_hbm.at[pl.ds(bt, _CH)])

    return sc_gather


@functools.lru_cache(maxsize=None)
def _build_sc_scatter():
    mesh = plsc.VectorSubcoreMesh(core_axis_name="c", subcore_axis_name="s")

    P = 2
    NRING = (_NCHUNK // P) * P

    @functools.partial(
        pl.kernel, mesh=mesh,
        out_type=jax.ShapeDtypeStruct((_NC, _NPAD, D), jnp.float32),
        scratch_types=[pltpu.VMEM((_NCHUNK, _CH), jnp.int32)]
                      + [pltpu.VMEM((_CH, D), jnp.float32)] * P
                      + [pltpu.VMEM_SHARED((_NPAD, D), jnp.float32)]
                      + [pltpu.SemaphoreType.DMA] * (2 * P),
    )
    def sc_scatter(enew_hbm, idx1_hbm, zeros_hbm, out_hbm, idx_v, *rest):
        rbuf = rest[0:P]
        acc_sh = rest[P]
        sr = rest[P + 1:2 * P + 1]
        sa = rest[2 * P + 1:3 * P + 1]
        cid = lax.axis_index("c")
        sid = lax.axis_index("s")
        wid = sid * _NC + cid
        pltpu.sync_copy(zeros_hbm, acc_sh.at[pl.ds(sid * _STRIPE, _STRIPE)])
        plsc.subcore_barrier()
        pltpu.sync_copy(idx1_hbm.at[wid], idx_v)

        def start_read(c, u):
            b = wid * _EPW + c * _CH
            pltpu.async_copy(enew_hbm.at[pl.ds(b, _CH)], rbuf[u], sr[u])

        def wait_read(c, u):
            b = wid * _EPW + c * _CH
            pltpu.make_async_copy(enew_hbm.at[pl.ds(b, _CH)], rbuf[u],
                                  sr[u]).wait()

        for u in range(P):
            start_read(u, u)

        def body(k, carry):
            for u in range(P):
                c = P * k + u
                wait_read(c, u)
                pltpu.async_copy(rbuf[u], acc_sh.at[idx_v.at[c]], sa[u],
                                 add=True)
            for u in range(P):
                c = P * k + u
                cn = c + P
                pltpu.make_async_copy(rbuf[u], acc_sh.at[idx_v.at[c]],
                                      sa[u]).wait()

                @pl.when(cn < NRING)
                def _():
                    start_read(cn, u)
            return carry

        lax.fori_loop(0, NRING // P, body, 0)
        for ct in range(NRING, _NCHUNK):
            start_read(ct, 0)
            wait_read(ct, 0)
            pltpu.sync_copy(rbuf[0], acc_sh.at[idx_v.at[ct]], add=True)
        plsc.subcore_barrier()
        pltpu.sync_copy(acc_sh.at[pl.ds(sid * _STRIPE, _STRIPE)],
                        out_hbm.at[cid, pl.ds(sid * _STRIPE, _STRIPE)])

    return sc_scatter


def _sc_gather(gs, md, idx0, idx1):
    return _build_sc_gather()(gs, md, idx0, idx1)


def _sc_scatter(e_new, idx1, zeros):
    return _build_sc_scatter()(e_new, idx1, zeros)


def kernel(grid_nfeat, mesh_nfeat, edge_index, grid2mesh_efeat, params):
    p = params

    def vec(w, name):
        return w[name].reshape(1, -1)

    in_w1 = p["in_edge"]["W1"]
    w1e, w1s, w1d = in_w1[0:D], in_w1[D:2 * D], in_w1[2 * D:3 * D]
    in_node_w1 = p["in_node"]["W1"]
    wa, wm = in_node_w1[0:D], in_node_w1[D:2 * D]

    ge = p["grid_emb"]
    gn = p["grid_node"]
    me = p["mesh_emb"]
    g_out, gs, m, md = _run_rows(
        _gm_body, 25, 400,
        [grid_nfeat, mesh_nfeat,
         ge["W1"], vec(ge, "b1"), ge["W2"], vec(ge, "b2"),
         vec(ge, "g"), vec(ge, "bt"),
         gn["W1"], vec(gn, "b1"), gn["W2"], vec(gn, "b2"),
         vec(gn, "g"), vec(gn, "bt"), w1s,
         me["W1"], vec(me, "b1"), me["W2"], vec(me, "b2"),
         vec(me, "g"), vec(me, "bt"), w1d],
        [(D, jnp.float32)] * 4, n_blocked=2)

    idx0 = edge_index[0].reshape(_NW, _NCHUNK, _CH)
    idx1 = edge_index[1].reshape(_NW, _NCHUNK, _CH)
    sump = _sc_gather(gs, md, idx0, idx1)

    ee = p["edge_emb"]
    ie = p["in_edge"]
    e_out, e_new = _run_rows(
        _edge_body, 160, 2000,
        [grid2mesh_efeat, sump,
         ee["W1"], vec(ee, "b1"), ee["W2"], vec(ee, "b2"),
         vec(ee, "g"), vec(ee, "bt"),
         w1e, vec(ie, "b1"), ie["W2"], vec(ie, "b2"),
         vec(ie, "g"), vec(ie, "bt")],
        [(D, jnp.float32), (D, jnp.float32)], n_blocked=2)

    zeros = jnp.zeros((_STRIPE, D), jnp.float32)
    partials = _sc_scatter(e_new, idx1, zeros)
    p0 = partials[0, :N_MESH]
    p1 = partials[1, :N_MESH]

    inn = p["in_node"]
    m_out = _run_rows(
        _node_body, 25, 400,
        [m, p0, p1, wa, wm, vec(inn, "b1"), inn["W2"], vec(inn, "b2"),
         vec(inn, "g"), vec(inn, "bt")],
        [(D, jnp.float32)], n_blocked=3)

    return (g_out, m_out, e_out)


# speedup vs baseline: 5.0654x; 1.0828x over previous
import functools

import jax
import jax.numpy as jnp
from jax import lax
from jax.experimental import pallas as pl
from jax.experimental.pallas import tpu as pltpu
from jax.experimental.pallas import tpu_sc as plsc

D = 128
N_GRID = 10000
N_MESH = 10000
E = 320000

_NC = 2
_NS = 16
_NW = _NC * _NS
_EPW = E // _NW
_CH = 80
_NCHUNK = _EPW // _CH
_NPAD = 10240
_STRIPE = _NPAD // _NS


def _silu(x):
    return x * jax.nn.sigmoid(x)


def _ln(y, g, bt):
    mu = jnp.mean(y, axis=-1, keepdims=True)
    yc = y - mu
    var = jnp.mean(yc * yc, axis=-1, keepdims=True)
    return yc * lax.rsqrt(var + 1e-5) * g + bt


def _full_spec(a):
    nd = a.ndim
    return pl.BlockSpec(a.shape, lambda i, _n=nd: (0,) * _n)


def _row_spec(rows, cols):
    return pl.BlockSpec((rows, cols), lambda i: (i, 0))


def _gm_body(xg, xm,
             gw1, gb1, gw2, gb2, gg, gbt,
             nw1, nb1, nw2, nb2, ng, nbt, ws,
             mw1, mb1, mw2, mb2, mg, mbt, wd,
             gout_ref, gs_ref, m_ref, md_ref):
    h = _silu(jnp.dot(xg[...], gw1[...], preferred_element_type=jnp.float32)
              + gb1[...])
    g = _ln(jnp.dot(h, gw2[...], preferred_element_type=jnp.float32)
            + gb2[...], gg[...], gbt[...])
    h2 = _silu(jnp.dot(g, nw1[...], preferred_element_type=jnp.float32)
               + nb1[...])
    y2 = jnp.dot(h2, nw2[...], preferred_element_type=jnp.float32) + nb2[...]
    gout_ref[...] = g + _ln(y2, ng[...], nbt[...])
    gs_ref[...] = jnp.dot(g, ws[...], preferred_element_type=jnp.float32)
    hm = _silu(jnp.dot(xm[...], mw1[...], preferred_element_type=jnp.float32)
               + mb1[...])
    m = _ln(jnp.dot(hm, mw2[...], preferred_element_type=jnp.float32)
            + mb2[...], mg[...], mbt[...])
    m_ref[...] = m
    md_ref[...] = jnp.dot(m, wd[...], preferred_element_type=jnp.float32)


def _edge_body(eft, sump, ew1, eb1, ew2, eb2, eg, ebt,
               we, ib1, iw2, ib2, ig, ibt, eout_ref, enew_ref):
    h0pre = jax.lax.dot_general(
        eft[...], ew1[...], (((0,), (0,)), ((), ())),
        preferred_element_type=jnp.float32)
    h0 = _silu(h0pre + eb1[...])
    e = _ln(jnp.dot(h0, ew2[...], preferred_element_type=jnp.float32)
            + eb2[...], eg[...], ebt[...])
    pre = (jnp.dot(e, we[...], preferred_element_type=jnp.float32) + ib1[...]
           + sump[...])
    h = _silu(pre)
    en = _ln(jnp.dot(h, iw2[...], preferred_element_type=jnp.float32)
             + ib2[...], ig[...], ibt[...])
    enew_ref[...] = en
    eout_ref[...] = e + en


def _node_body(m, p0, p1, wa, wm, b1, w2, b2, gg, bt, mout_ref):
    agg = p0[...] + p1[...]
    h = _silu(jnp.dot(agg, wa[...], preferred_element_type=jnp.float32)
              + jnp.dot(m[...], wm[...], preferred_element_type=jnp.float32)
              + b1[...])
    mn = _ln(jnp.dot(h, w2[...], preferred_element_type=jnp.float32) + b2[...],
             gg[...], bt[...])
    mout_ref[...] = m[...] + mn


def _run_rows(body, grid_n, row_block, ins, outs, n_blocked=1):
    out_shape = tuple(jax.ShapeDtypeStruct((grid_n * row_block, c), dt)
                      for c, dt in outs)
    in_specs = [_row_spec(row_block, a.shape[-1]) if k < n_blocked
                else _full_spec(a) for k, a in enumerate(ins)]
    out_specs = tuple(_row_spec(row_block, c) for c, _ in outs)
    one = len(outs) == 1
    return pl.pallas_call(
        body,
        grid=(grid_n,),
        in_specs=in_specs,
        out_specs=out_specs[0] if one else out_specs,
        out_shape=out_shape[0] if one else out_shape,
    )(*ins)


@functools.lru_cache(maxsize=None)
def _build_sc_gather():
    mesh = plsc.VectorSubcoreMesh(core_axis_name="c", subcore_axis_name="s")
    P = 4
    NRING = (_NCHUNK // P) * P

    @functools.partial(
        pl.kernel, mesh=mesh,
        out_type=jax.ShapeDtypeStruct((E, D), jnp.float32),
        scratch_types=[pltpu.VMEM((_NCHUNK, _CH), jnp.int32),
                       pltpu.VMEM((_NCHUNK, _CH), jnp.int32)]
                      + [pltpu.VMEM((_CH, D), jnp.float32)] * (2 * P)
                      + [pltpu.SemaphoreType.DMA] * (3 * P),
    )
    def sc_gather(gs_hbm, md_hbm, idx0_hbm, idx1_hbm, sum_hbm,
                  idx0_v, idx1_v, *bufsems):
        ra = bufsems[0:P]
        rb = bufsems[P:2 * P]
        sga = bufsems[2 * P:3 * P]
        sgb = bufsems[3 * P:4 * P]
        sw = bufsems[4 * P:5 * P]
        wid = lax.axis_index("s") * _NC + lax.axis_index("c")
        pltpu.sync_copy(idx0_hbm.at[wid], idx0_v)
        pltpu.sync_copy(idx1_hbm.at[wid], idx1_v)

        def add_into(dst, src):
            def add_body(i, carry):
                for q in range(D // 16):
                    o = q * 16
                    dst[i, pl.ds(o, 16)] = (dst[i, pl.ds(o, 16)]
                                            + src[i, pl.ds(o, 16)])
                return carry
            lax.fori_loop(0, _CH, add_body, 0)

        def start(c, u):
            pltpu.async_copy(gs_hbm.at[idx0_v.at[c]], ra[u], sga[u])
            pltpu.async_copy(md_hbm.at[idx1_v.at[c]], rb[u], sgb[u])

        def wait_gather(c, u):
            pltpu.make_async_copy(gs_hbm.at[idx0_v.at[c]], ra[u],
                                  sga[u]).wait()
            pltpu.make_async_copy(md_hbm.at[idx1_v.at[c]], rb[u],
                                  sgb[u]).wait()

        for u in range(P):
            start(u, u)

        def body(k, carry):
            for u in range(P):
                c = P * k + u
                b = wid * _EPW + c * _CH
                wait_gather(c, u)
                add_into(ra[u], rb[u])
                pltpu.async_copy(ra[u], sum_hbm.at[pl.ds(b, _CH)], sw[u])
            for u in range(P):
                c = P * k + u
                cn = c + P
                b = wid * _EPW + c * _CH
                pltpu.make_async_copy(ra[u], sum_hbm.at[pl.ds(b, _CH)],
                                      sw[u]).wait()

                @pl.when(cn < NRING)
                def _():
                    start(cn, u)
            return carry

        lax.fori_loop(0, NRING // P, body, 0)
        for ct in range(NRING, _NCHUNK):
            bt = wid * _EPW + ct * _CH
            start(ct, 0)
            wait_gather(ct, 0)
            add_into(ra[0], rb[0])
            pltpu.sync_copy(ra[0], sum_hbm.at[pl.ds(bt, _CH)])

    return sc_gather


@functools.lru_cache(maxsize=None)
def _build_sc_scatter():
    mesh = plsc.VectorSubcoreMesh(core_axis_name="c", subcore_axis_name="s")

    P = 2
    NRING = (_NCHUNK // P) * P

    @functools.partial(
        pl.kernel, mesh=mesh,
        out_type=jax.ShapeDtypeStruct((_NC, _NPAD, D), jnp.float32),
        scratch_types=[pltpu.VMEM((_NCHUNK, _CH), jnp.int32)]
                      + [pltpu.VMEM((_CH, D), jnp.float32)] * P
                      + [pltpu.VMEM_SHARED((_NPAD, D), jnp.float32)]
                      + [pltpu.SemaphoreType.DMA] * (2 * P),
    )
    def sc_scatter(enew_hbm, idx1_hbm, zeros_hbm, out_hbm, idx_v, *rest):
        rbuf = rest[0:P]
        acc_sh = rest[P]
        sr = rest[P + 1:2 * P + 1]
        sa = rest[2 * P + 1:3 * P + 1]
        cid = lax.axis_index("c")
        sid = lax.axis_index("s")
        wid = sid * _NC + cid
        pltpu.sync_copy(zeros_hbm, acc_sh.at[pl.ds(sid * _STRIPE, _STRIPE)])
        plsc.subcore_barrier()
        pltpu.sync_copy(idx1_hbm.at[wid], idx_v)

        def start_read(c, u):
            b = wid * _EPW + c * _CH
            pltpu.async_copy(enew_hbm.at[pl.ds(b, _CH)], rbuf[u], sr[u])

        def wait_read(c, u):
            b = wid * _EPW + c * _CH
            pltpu.make_async_copy(enew_hbm.at[pl.ds(b, _CH)], rbuf[u],
                                  sr[u]).wait()

        for u in range(P):
            start_read(u, u)

        def body(k, carry):
            for u in range(P):
                c = P * k + u
                wait_read(c, u)
                pltpu.async_copy(rbuf[u], acc_sh.at[idx_v.at[c]], sa[u],
                                 add=True)
            for u in range(P):
                c = P * k + u
                cn = c + P
                pltpu.make_async_copy(rbuf[u], acc_sh.at[idx_v.at[c]],
                                      sa[u]).wait()

                @pl.when(cn < NRING)
                def _():
                    start_read(cn, u)
            return carry

        lax.fori_loop(0, NRING // P, body, 0)
        for ct in range(NRING, _NCHUNK):
            start_read(ct, 0)
            wait_read(ct, 0)
            pltpu.sync_copy(rbuf[0], acc_sh.at[idx_v.at[ct]], add=True)
        plsc.subcore_barrier()
        pltpu.sync_copy(acc_sh.at[pl.ds(sid * _STRIPE, _STRIPE)],
                        out_hbm.at[cid, pl.ds(sid * _STRIPE, _STRIPE)])

    return sc_scatter


def _sc_gather(gs, md, idx0, idx1):
    return _build_sc_gather()(gs, md, idx0, idx1)


def _sc_scatter(e_new, idx1, zeros):
    return _build_sc_scatter()(e_new, idx1, zeros)


def kernel(grid_nfeat, mesh_nfeat, edge_index, grid2mesh_efeat, params):
    p = params

    def vec(w, name):
        return w[name].reshape(1, -1)

    in_w1 = p["in_edge"]["W1"]
    w1e, w1s, w1d = in_w1[0:D], in_w1[D:2 * D], in_w1[2 * D:3 * D]
    in_node_w1 = p["in_node"]["W1"]
    wa, wm = in_node_w1[0:D], in_node_w1[D:2 * D]

    ge = p["grid_emb"]
    gn = p["grid_node"]
    me = p["mesh_emb"]
    g_out, gs, m, md = _run_rows(
        _gm_body, 25, 400,
        [grid_nfeat, mesh_nfeat,
         ge["W1"], vec(ge, "b1"), ge["W2"], vec(ge, "b2"),
         vec(ge, "g"), vec(ge, "bt"),
         gn["W1"], vec(gn, "b1"), gn["W2"], vec(gn, "b2"),
         vec(gn, "g"), vec(gn, "bt"), w1s,
         me["W1"], vec(me, "b1"), me["W2"], vec(me, "b2"),
         vec(me, "g"), vec(me, "bt"), w1d],
        [(D, jnp.float32)] * 4, n_blocked=2)

    idx0 = edge_index[0].reshape(_NW, _NCHUNK, _CH)
    idx1 = edge_index[1].reshape(_NW, _NCHUNK, _CH)
    sump = _sc_gather(gs, md, idx0, idx1)

    ee = p["edge_emb"]
    ie = p["in_edge"]
    RE = 3200
    NEB = E // RE
    eins = [grid2mesh_efeat.T, sump,
            ee["W1"], vec(ee, "b1"), ee["W2"], vec(ee, "b2"),
            vec(ee, "g"), vec(ee, "bt"),
            w1e, vec(ie, "b1"), ie["W2"], vec(ie, "b2"),
            vec(ie, "g"), vec(ie, "bt")]
    e_out, e_new = pl.pallas_call(
        _edge_body,
        grid=(NEB,),
        in_specs=[pl.BlockSpec((4, RE), lambda i: (0, i)),
                  _row_spec(RE, D)] + [_full_spec(a) for a in eins[2:]],
        out_specs=(_row_spec(RE, D), _row_spec(RE, D)),
        out_shape=(jax.ShapeDtypeStruct((E, D), jnp.float32),
                   jax.ShapeDtypeStruct((E, D), jnp.float32)),
    )(*eins)

    zeros = jnp.zeros((_STRIPE, D), jnp.float32)
    partials = _sc_scatter(e_new, idx1, zeros)
    p0 = partials[0, :N_MESH]
    p1 = partials[1, :N_MESH]

    inn = p["in_node"]
    m_out = _run_rows(
        _node_body, 25, 400,
        [m, p0, p1, wa, wm, vec(inn, "b1"), inn["W2"], vec(inn, "b2"),
         vec(inn, "g"), vec(inn, "bt")],
        [(D, jnp.float32)], n_blocked=3)

    return (g_out, m_out, e_out)


# speedup vs baseline: 5.3298x; 1.0522x over previous
import functools

import jax
import jax.numpy as jnp
from jax import lax
from jax.experimental import pallas as pl
from jax.experimental.pallas import tpu as pltpu
from jax.experimental.pallas import tpu_sc as plsc

D = 128
N_GRID = 10000
N_MESH = 10000
E = 320000

_NC = 2
_NS = 16
_NW = _NC * _NS
_EPW = E // _NW
_CH = 80
_NCHUNK = _EPW // _CH
_NPAD = 10240
_STRIPE = _NPAD // _NS


def _silu(x):
    return x * jax.nn.sigmoid(x)


def _ln(y, g, bt):
    mu = jnp.mean(y, axis=-1, keepdims=True)
    yc = y - mu
    var = jnp.mean(yc * yc, axis=-1, keepdims=True)
    return yc * lax.rsqrt(var + 1e-5) * g + bt


def _pack_ilv(x):
    xb = x.astype(jnp.bfloat16)
    return jnp.stack([xb[:, :64], xb[:, 64:]], axis=-1).reshape(x.shape)


def _full_spec(a):
    nd = a.ndim
    return pl.BlockSpec(a.shape, lambda i, _n=nd: (0,) * _n)


def _row_spec(rows, cols):
    return pl.BlockSpec((rows, cols), lambda i: (i, 0))


def _gm_body(xg, xm,
             gw1, gb1, gw2, gb2, gg, gbt,
             nw1, nb1, nw2, nb2, ng, nbt, ws,
             mw1, mb1, mw2, mb2, mg, mbt, wd,
             gout_ref, gs_ref, m_ref, md_ref):
    h = _silu(jnp.dot(xg[...], gw1[...], preferred_element_type=jnp.float32)
              + gb1[...])
    g = _ln(jnp.dot(h, gw2[...], preferred_element_type=jnp.float32)
            + gb2[...], gg[...], gbt[...])
    h2 = _silu(jnp.dot(g, nw1[...], preferred_element_type=jnp.float32)
               + nb1[...])
    y2 = jnp.dot(h2, nw2[...], preferred_element_type=jnp.float32) + nb2[...]
    gout_ref[...] = g + _ln(y2, ng[...], nbt[...])
    gs_ref[...] = jnp.dot(g, ws[...], preferred_element_type=jnp.float32)
    hm = _silu(jnp.dot(xm[...], mw1[...], preferred_element_type=jnp.float32)
               + mb1[...])
    m = _ln(jnp.dot(hm, mw2[...], preferred_element_type=jnp.float32)
            + mb2[...], mg[...], mbt[...])
    m_ref[...] = m
    md_ref[...] = jnp.dot(m, wd[...], preferred_element_type=jnp.float32)


def _edge_body(eft, sump, ew1, eb1, ew2, eb2, eg, ebt,
               we, ib1, iw2, ib2, ig, ibt, eout_ref, enew_ref):
    h0pre = jax.lax.dot_general(
        eft[...], ew1[...], (((0,), (0,)), ((), ())),
        preferred_element_type=jnp.float32)
    h0 = _silu(h0pre + eb1[...])
    e = _ln(jnp.dot(h0, ew2[...], preferred_element_type=jnp.float32)
            + eb2[...], eg[...], ebt[...])
    pre = (jnp.dot(e, we[...], preferred_element_type=jnp.float32) + ib1[...]
           + sump[...])
    h = _silu(pre)
    en = _ln(jnp.dot(h, iw2[...], preferred_element_type=jnp.float32)
             + ib2[...], ig[...], ibt[...])
    enew_ref[...] = en
    eout_ref[...] = e + en


def _node_body(m, p0, p1, wa, wm, b1, w2, b2, gg, bt, mout_ref):
    agg = p0[0] + p1[0]
    h = _silu(jnp.dot(agg, wa[...], preferred_element_type=jnp.float32)
              + jnp.dot(m[...], wm[...], preferred_element_type=jnp.float32)
              + b1[...])
    mn = _ln(jnp.dot(h, w2[...], preferred_element_type=jnp.float32) + b2[...],
             gg[...], bt[...])
    mout_ref[...] = m[...] + mn


def _run_rows(body, grid_n, row_block, ins, outs, n_blocked=1):
    out_shape = tuple(jax.ShapeDtypeStruct((grid_n * row_block, c), dt)
                      for c, dt in outs)
    in_specs = [_row_spec(row_block, a.shape[-1]) if k < n_blocked
                else _full_spec(a) for k, a in enumerate(ins)]
    out_specs = tuple(_row_spec(row_block, c) for c, _ in outs)
    one = len(outs) == 1
    return pl.pallas_call(
        body,
        grid=(grid_n,),
        in_specs=in_specs,
        out_specs=out_specs[0] if one else out_specs,
        out_shape=out_shape[0] if one else out_shape,
    )(*ins)


@functools.lru_cache(maxsize=None)
def _build_sc_gather():
    mesh = plsc.VectorSubcoreMesh(core_axis_name="c", subcore_axis_name="s")
    P = 4
    NRING = (_NCHUNK // P) * P

    @functools.partial(
        pl.kernel, mesh=mesh,
        out_type=jax.ShapeDtypeStruct((E, D), jnp.float32),
        scratch_types=[pltpu.VMEM((_NCHUNK, _CH), jnp.int32),
                       pltpu.VMEM((_NCHUNK, _CH), jnp.int32)]
                      + [pltpu.VMEM((_CH, D), jnp.float32)] * (2 * P)
                      + [pltpu.SemaphoreType.DMA] * (3 * P),
    )
    def sc_gather(gs_hbm, md_hbm, idx0_hbm, idx1_hbm, sum_hbm,
                  idx0_v, idx1_v, *bufsems):
        ra = bufsems[0:P]
        rb = bufsems[P:2 * P]
        sga = bufsems[2 * P:3 * P]
        sgb = bufsems[3 * P:4 * P]
        sw = bufsems[4 * P:5 * P]
        wid = lax.axis_index("s") * _NC + lax.axis_index("c")
        pltpu.sync_copy(idx0_hbm.at[wid], idx0_v)
        pltpu.sync_copy(idx1_hbm.at[wid], idx1_v)

        def add_into(dst, src):
            def add_body(i, carry):
                for q in range(D // 16):
                    o = q * 16
                    dst[i, pl.ds(o, 16)] = (dst[i, pl.ds(o, 16)]
                                            + src[i, pl.ds(o, 16)])
                return carry
            lax.fori_loop(0, _CH, add_body, 0)

        def start(c, u):
            pltpu.async_copy(gs_hbm.at[idx0_v.at[c]], ra[u], sga[u])
            pltpu.async_copy(md_hbm.at[idx1_v.at[c]], rb[u], sgb[u])

        def wait_gather(c, u):
            pltpu.make_async_copy(gs_hbm.at[idx0_v.at[c]], ra[u],
                                  sga[u]).wait()
            pltpu.make_async_copy(md_hbm.at[idx1_v.at[c]], rb[u],
                                  sgb[u]).wait()

        for u in range(P):
            start(u, u)

        def body(k, carry):
            for u in range(P):
                c = P * k + u
                b = wid * _EPW + c * _CH
                wait_gather(c, u)
                add_into(ra[u], rb[u])
                pltpu.async_copy(ra[u], sum_hbm.at[pl.ds(b, _CH)], sw[u])
            for u in range(P):
                c = P * k + u
                cn = c + P
                b = wid * _EPW + c * _CH
                pltpu.make_async_copy(ra[u], sum_hbm.at[pl.ds(b, _CH)],
                                      sw[u]).wait()

                @pl.when(cn < NRING)
                def _():
                    start(cn, u)
            return carry

        lax.fori_loop(0, NRING // P, body, 0)
        for ct in range(NRING, _NCHUNK):
            bt = wid * _EPW + ct * _CH
            start(ct, 0)
            wait_gather(ct, 0)
            add_into(ra[0], rb[0])
            pltpu.sync_copy(ra[0], sum_hbm.at[pl.ds(bt, _CH)])

    return sc_gather


@functools.lru_cache(maxsize=None)
def _build_sc_scatter():
    mesh = plsc.VectorSubcoreMesh(core_axis_name="c", subcore_axis_name="s")

    P = 3
    NRING = (_NCHUNK // P) * P

    @functools.partial(
        pl.kernel, mesh=mesh,
        out_type=jax.ShapeDtypeStruct((_NC, _NPAD, D), jnp.float32),
        scratch_types=[pltpu.VMEM((1, _CH), jnp.int32)] * P
                      + [pltpu.VMEM((_CH, D), jnp.float32)] * P
                      + [pltpu.VMEM_SHARED((_NPAD, D), jnp.float32)]
                      + [pltpu.SemaphoreType.DMA] * (3 * P),
    )
    def sc_scatter(enew_hbm, idx1_hbm, zeros_hbm, out_hbm, *rest):
        ibuf = rest[0:P]
        rbuf = rest[P:2 * P]
        acc_sh = rest[2 * P]
        sr = rest[2 * P + 1:3 * P + 1]
        si = rest[3 * P + 1:4 * P + 1]
        sa = rest[4 * P + 1:5 * P + 1]
        cid = lax.axis_index("c")
        sid = lax.axis_index("s")
        wid = sid * _NC + cid
        pltpu.sync_copy(zeros_hbm, acc_sh.at[pl.ds(sid * _STRIPE, _STRIPE)])
        plsc.subcore_barrier()

        def start_read(c, u):
            b = wid * _EPW + c * _CH
            pltpu.async_copy(enew_hbm.at[pl.ds(b, _CH)], rbuf[u], sr[u])

        def start_idx(c, u):
            pltpu.async_copy(idx1_hbm.at[wid].at[pl.ds(c, 1)], ibuf[u],
                             si[u])

        def wait_read(c, u):
            b = wid * _EPW + c * _CH
            pltpu.make_async_copy(enew_hbm.at[pl.ds(b, _CH)], rbuf[u],
                                  sr[u]).wait()
            pltpu.make_async_copy(idx1_hbm.at[wid].at[pl.ds(c, 1)], ibuf[u],
                                  si[u]).wait()

        for u in range(P):
            start_read(u, u)
            start_idx(u, u)

        def body(k, carry):
            for u in range(P):
                c = P * k + u
                wait_read(c, u)
                pltpu.async_copy(rbuf[u], acc_sh.at[ibuf[u].at[0]], sa[u],
                                 add=True)
            for u in range(P):
                c = P * k + u
                pltpu.make_async_copy(rbuf[u], acc_sh.at[ibuf[u].at[0]],
                                      sa[u]).wait()

                @pl.when(c + P < NRING)
                def _():
                    start_read(c + P, u)
                    start_idx(c + P, u)
            return carry

        lax.fori_loop(0, NRING // P, body, 0)
        for ct in range(NRING, _NCHUNK):
            start_read(ct, 0)
            start_idx(ct, 0)
            wait_read(ct, 0)
            pltpu.sync_copy(rbuf[0], acc_sh.at[ibuf[0].at[0]], add=True)
        plsc.subcore_barrier()
        pltpu.sync_copy(acc_sh.at[pl.ds(sid * _STRIPE, _STRIPE)],
                        out_hbm.at[cid, pl.ds(sid * _STRIPE, _STRIPE)])

    return sc_scatter


def _sc_gather(gs, md, idx0, idx1):
    return _build_sc_gather()(gs, md, idx0, idx1)


def _sc_scatter(e_new, idx1, zeros):
    return _build_sc_scatter()(e_new, idx1, zeros)


def kernel(grid_nfeat, mesh_nfeat, edge_index, grid2mesh_efeat, params):
    p = params

    def vec(w, name):
        return w[name].reshape(1, -1)

    in_w1 = p["in_edge"]["W1"]
    w1e, w1s, w1d = in_w1[0:D], in_w1[D:2 * D], in_w1[2 * D:3 * D]
    in_node_w1 = p["in_node"]["W1"]
    wa, wm = in_node_w1[0:D], in_node_w1[D:2 * D]

    ge = p["grid_emb"]
    gn = p["grid_node"]
    me = p["mesh_emb"]
    g_out, gs, m, md = _run_rows(
        _gm_body, 25, 400,
        [grid_nfeat, mesh_nfeat,
         ge["W1"], vec(ge, "b1"), ge["W2"], vec(ge, "b2"),
         vec(ge, "g"), vec(ge, "bt"),
         gn["W1"], vec(gn, "b1"), gn["W2"], vec(gn, "b2"),
         vec(gn, "g"), vec(gn, "bt"), w1s,
         me["W1"], vec(me, "b1"), me["W2"], vec(me, "b2"),
         vec(me, "g"), vec(me, "bt"), w1d],
        [(D, jnp.float32)] * 4, n_blocked=2)

    idx0 = edge_index[0].reshape(_NW, _NCHUNK, _CH)
    idx1 = edge_index[1].reshape(_NW, _NCHUNK, _CH)
    sump = _sc_gather(gs, md, idx0, idx1)

    ee = p["edge_emb"]
    ie = p["in_edge"]
    RE = 3200
    NEB = E // RE
    eins = [grid2mesh_efeat.T, sump,
            ee["W1"], vec(ee, "b1"), ee["W2"], vec(ee, "b2"),
            vec(ee, "g"), vec(ee, "bt"),
            w1e, vec(ie, "b1"), ie["W2"], vec(ie, "b2"),
            vec(ie, "g"), vec(ie, "bt")]
    e_out, e_new = pl.pallas_call(
        _edge_body,
        grid=(NEB,),
        in_specs=[pl.BlockSpec((4, RE), lambda i: (0, i)),
                  _row_spec(RE, D)] + [_full_spec(a) for a in eins[2:]],
        out_specs=(_row_spec(RE, D), _row_spec(RE, D)),
        out_shape=(jax.ShapeDtypeStruct((E, D), jnp.float32),
                   jax.ShapeDtypeStruct((E, D), jnp.float32)),
    )(*eins)

    zeros = jnp.zeros((_STRIPE, D), jnp.float32)
    partials = _sc_scatter(e_new, idx1, zeros)

    inn = p["in_node"]
    nins = [m, partials, partials, wa, wm, vec(inn, "b1"), inn["W2"],
            vec(inn, "b2"), vec(inn, "g"), vec(inn, "bt")]
    m_out = pl.pallas_call(
        _node_body,
        grid=(25,),
        in_specs=[_row_spec(400, D),
                  pl.BlockSpec((1, 400, D), lambda i: (0, i, 0)),
                  pl.BlockSpec((1, 400, D), lambda i: (1, i, 0))]
                 + [_full_spec(a) for a in nins[3:]],
        out_specs=_row_spec(400, D),
        out_shape=jax.ShapeDtypeStruct((N_MESH, D), jnp.float32),
    )(*nins)

    return (g_out, m_out, e_out)


# speedup vs baseline: 5.6576x; 1.0615x over previous
import functools

import jax
import jax.numpy as jnp
from jax import lax
from jax.experimental import pallas as pl
from jax.experimental.pallas import tpu as pltpu
from jax.experimental.pallas import tpu_sc as plsc

D = 128
N_GRID = 10000
N_MESH = 10000
E = 320000

_NC = 2
_NS = 16
_NW = _NC * _NS
_EPW = E // _NW
_CH = 80
_NCHUNK = _EPW // _CH
_NPAD = 10240
_STRIPE = _NPAD // _NS


def _silu(x):
    return x * jax.nn.sigmoid(x)


def _ln(y, g, bt):
    mu = jnp.mean(y, axis=-1, keepdims=True)
    yc = y - mu
    var = jnp.mean(yc * yc, axis=-1, keepdims=True)
    return yc * lax.rsqrt(var + 1e-5) * g + bt


def _pack_ilv(x):
    xb = x.astype(jnp.bfloat16)
    return jnp.stack([xb[:, :64], xb[:, 64:]], axis=-1).reshape(x.shape)


def _full_spec(a):
    nd = a.ndim
    return pl.BlockSpec(a.shape, lambda i, _n=nd: (0,) * _n)


def _row_spec(rows, cols):
    return pl.BlockSpec((rows, cols), lambda i: (i, 0))


def _gm_body(xg, xm,
             gw1, gb1, gw2, gb2, gg, gbt,
             nw1, nb1, nw2, nb2, ng, nbt, ws,
             mw1, mb1, mw2, mb2, mg, mbt, wd,
             gout_ref, gs_ref, m_ref, md_ref):
    h = _silu(jnp.dot(xg[...], gw1[...], preferred_element_type=jnp.float32)
              + gb1[...])
    g = _ln(jnp.dot(h, gw2[...], preferred_element_type=jnp.float32)
            + gb2[...], gg[...], gbt[...])
    h2 = _silu(jnp.dot(g, nw1[...], preferred_element_type=jnp.float32)
               + nb1[...])
    y2 = jnp.dot(h2, nw2[...], preferred_element_type=jnp.float32) + nb2[...]
    gout_ref[...] = g + _ln(y2, ng[...], nbt[...])
    gs_ref[...] = jnp.dot(g, ws[...], preferred_element_type=jnp.float32)
    hm = _silu(jnp.dot(xm[...], mw1[...], preferred_element_type=jnp.float32)
               + mb1[...])
    m = _ln(jnp.dot(hm, mw2[...], preferred_element_type=jnp.float32)
            + mb2[...], mg[...], mbt[...])
    m_ref[...] = m
    md_ref[...] = jnp.dot(m, wd[...], preferred_element_type=jnp.float32)


def _edge_body(eft, sump, ew1, eb1, ew2, eb2, eg, ebt,
               we, ib1, iw2, ib2, ig, ibt, eout_ref, enew_ref):
    h0pre = jax.lax.dot_general(
        eft[...], ew1[...], (((0,), (0,)), ((), ())),
        preferred_element_type=jnp.float32)
    h0 = _silu(h0pre + eb1[...])
    e = _ln(jnp.dot(h0, ew2[...], preferred_element_type=jnp.float32)
            + eb2[...], eg[...], ebt[...])
    pre = (jnp.dot(e, we[...], preferred_element_type=jnp.float32) + ib1[...]
           + sump[...])
    h = _silu(pre)
    en = _ln(jnp.dot(h, iw2[...], preferred_element_type=jnp.float32)
             + ib2[...], ig[...], ibt[...])
    enew_ref[...] = en
    eout_ref[...] = e + en


def _node_body(m, pa0, pa1, pb0, pb1, wa, wm, b1, w2, b2, gg, bt, mout_ref):
    agg = pa0[0] + pa1[0] + pb0[0] + pb1[0]
    h = _silu(jnp.dot(agg, wa[...], preferred_element_type=jnp.float32)
              + jnp.dot(m[...], wm[...], preferred_element_type=jnp.float32)
              + b1[...])
    mn = _ln(jnp.dot(h, w2[...], preferred_element_type=jnp.float32) + b2[...],
             gg[...], bt[...])
    mout_ref[...] = m[...] + mn


def _run_rows(body, grid_n, row_block, ins, outs, n_blocked=1):
    out_shape = tuple(jax.ShapeDtypeStruct((grid_n * row_block, c), dt)
                      for c, dt in outs)
    in_specs = [_row_spec(row_block, a.shape[-1]) if k < n_blocked
                else _full_spec(a) for k, a in enumerate(ins)]
    out_specs = tuple(_row_spec(row_block, c) for c, _ in outs)
    one = len(outs) == 1
    return pl.pallas_call(
        body,
        grid=(grid_n,),
        in_specs=in_specs,
        out_specs=out_specs[0] if one else out_specs,
        out_shape=out_shape[0] if one else out_shape,
    )(*ins)


@functools.lru_cache(maxsize=None)
def _build_sc_gather(nch):
    mesh = plsc.VectorSubcoreMesh(core_axis_name="c", subcore_axis_name="s")
    P = 4
    NRING = (nch // P) * P
    epw = nch * _CH

    @functools.partial(
        pl.kernel, mesh=mesh,
        out_type=jax.ShapeDtypeStruct((_NW * epw, D), jnp.float32),
        scratch_types=[pltpu.VMEM((nch, _CH), jnp.int32),
                       pltpu.VMEM((nch, _CH), jnp.int32)]
                      + [pltpu.VMEM((_CH, D), jnp.float32)] * (2 * P)
                      + [pltpu.SemaphoreType.DMA] * (3 * P),
    )
    def sc_gather(gs_hbm, md_hbm, idx0_hbm, idx1_hbm, sum_hbm,
                  idx0_v, idx1_v, *bufsems):
        ra = bufsems[0:P]
        rb = bufsems[P:2 * P]
        sga = bufsems[2 * P:3 * P]
        sgb = bufsems[3 * P:4 * P]
        sw = bufsems[4 * P:5 * P]
        wid = lax.axis_index("s") * _NC + lax.axis_index("c")
        pltpu.sync_copy(idx0_hbm.at[wid], idx0_v)
        pltpu.sync_copy(idx1_hbm.at[wid], idx1_v)

        def add_into(dst, src):
            def add_body(i, carry):
                for q in range(D // 16):
                    o = q * 16
                    dst[i, pl.ds(o, 16)] = (dst[i, pl.ds(o, 16)]
                                            + src[i, pl.ds(o, 16)])
                return carry
            lax.fori_loop(0, _CH, add_body, 0)

        def start(c, u):
            pltpu.async_copy(gs_hbm.at[idx0_v.at[c]], ra[u], sga[u])
            pltpu.async_copy(md_hbm.at[idx1_v.at[c]], rb[u], sgb[u])

        def wait_gather(c, u):
            pltpu.make_async_copy(gs_hbm.at[idx0_v.at[c]], ra[u],
                                  sga[u]).wait()
            pltpu.make_async_copy(md_hbm.at[idx1_v.at[c]], rb[u],
                                  sgb[u]).wait()

        for u in range(P):
            start(u, u)

        def body(k, carry):
            for u in range(P):
                c = P * k + u
                b = wid * epw + c * _CH
                wait_gather(c, u)
                add_into(ra[u], rb[u])
                pltpu.async_copy(ra[u], sum_hbm.at[pl.ds(b, _CH)], sw[u])
            for u in range(P):
                c = P * k + u
                cn = c + P
                b = wid * epw + c * _CH
                pltpu.make_async_copy(ra[u], sum_hbm.at[pl.ds(b, _CH)],
                                      sw[u]).wait()

                @pl.when(cn < NRING)
                def _():
                    start(cn, u)
            return carry

        lax.fori_loop(0, NRING // P, body, 0)
        for ct in range(NRING, nch):
            bt = wid * epw + ct * _CH
            start(ct, 0)
            wait_gather(ct, 0)
            add_into(ra[0], rb[0])
            pltpu.sync_copy(ra[0], sum_hbm.at[pl.ds(bt, _CH)])

    return sc_gather


@functools.lru_cache(maxsize=None)
def _build_sc_scatter(e0, nch):
    mesh = plsc.VectorSubcoreMesh(core_axis_name="c", subcore_axis_name="s")

    P = 3
    NRING = (nch // P) * P
    epw = nch * _CH

    @functools.partial(
        pl.kernel, mesh=mesh,
        out_type=jax.ShapeDtypeStruct((_NC, _NPAD, D), jnp.float32),
        scratch_types=[pltpu.VMEM((1, _CH), jnp.int32)] * P
                      + [pltpu.VMEM((_CH, D), jnp.float32)] * P
                      + [pltpu.VMEM_SHARED((_NPAD, D), jnp.float32)]
                      + [pltpu.SemaphoreType.DMA] * (3 * P),
    )
    def sc_scatter(enew_hbm, idx1_hbm, zeros_hbm, out_hbm, *rest):
        ibuf = rest[0:P]
        rbuf = rest[P:2 * P]
        acc_sh = rest[2 * P]
        sr = rest[2 * P + 1:3 * P + 1]
        si = rest[3 * P + 1:4 * P + 1]
        sa = rest[4 * P + 1:5 * P + 1]
        cid = lax.axis_index("c")
        sid = lax.axis_index("s")
        wid = sid * _NC + cid
        pltpu.sync_copy(zeros_hbm, acc_sh.at[pl.ds(sid * _STRIPE, _STRIPE)])
        plsc.subcore_barrier()

        def start_read(c, u):
            b = e0 + wid * epw + c * _CH
            pltpu.async_copy(enew_hbm.at[pl.ds(b, _CH)], rbuf[u], sr[u])

        def start_idx(c, u):
            pltpu.async_copy(idx1_hbm.at[wid].at[pl.ds(c, 1)], ibuf[u],
                             si[u])

        def wait_read(c, u):
            b = e0 + wid * epw + c * _CH
            pltpu.make_async_copy(enew_hbm.at[pl.ds(b, _CH)], rbuf[u],
                                  sr[u]).wait()
            pltpu.make_async_copy(idx1_hbm.at[wid].at[pl.ds(c, 1)], ibuf[u],
                                  si[u]).wait()

        for u in range(P):
            start_read(u, u)
            start_idx(u, u)

        def body(k, carry):
            for u in range(P):
                c = P * k + u
                wait_read(c, u)
                pltpu.async_copy(rbuf[u], acc_sh.at[ibuf[u].at[0]], sa[u],
                                 add=True)
            for u in range(P):
                c = P * k + u
                pltpu.make_async_copy(rbuf[u], acc_sh.at[ibuf[u].at[0]],
                                      sa[u]).wait()

                @pl.when(c + P < NRING)
                def _():
                    start_read(c + P, u)
                    start_idx(c + P, u)
            return carry

        lax.fori_loop(0, NRING // P, body, 0)
        for ct in range(NRING, nch):
            start_read(ct, 0)
            start_idx(ct, 0)
            wait_read(ct, 0)
            pltpu.sync_copy(rbuf[0], acc_sh.at[ibuf[0].at[0]], add=True)
        plsc.subcore_barrier()
        pltpu.sync_copy(acc_sh.at[pl.ds(sid * _STRIPE, _STRIPE)],
                        out_hbm.at[cid, pl.ds(sid * _STRIPE, _STRIPE)])

    return sc_scatter


def _sc_gather(gs, md, idx0, idx1, nch):
    return _build_sc_gather(nch)(gs, md, idx0, idx1)


def _sc_scatter(e_new, idx1, zeros, e0, nch):
    return _build_sc_scatter(e0, nch)(e_new, idx1, zeros)


def kernel(grid_nfeat, mesh_nfeat, edge_index, grid2mesh_efeat, params):
    p = params

    def vec(w, name):
        return w[name].reshape(1, -1)

    in_w1 = p["in_edge"]["W1"]
    w1e, w1s, w1d = in_w1[0:D], in_w1[D:2 * D], in_w1[2 * D:3 * D]
    in_node_w1 = p["in_node"]["W1"]
    wa, wm = in_node_w1[0:D], in_node_w1[D:2 * D]

    ge = p["grid_emb"]
    gn = p["grid_node"]
    me = p["mesh_emb"]
    g_out, gs, m, md = _run_rows(
        _gm_body, 25, 400,
        [grid_nfeat, mesh_nfeat,
         ge["W1"], vec(ge, "b1"), ge["W2"], vec(ge, "b2"),
         vec(ge, "g"), vec(ge, "bt"),
         gn["W1"], vec(gn, "b1"), gn["W2"], vec(gn, "b2"),
         vec(gn, "g"), vec(gn, "bt"), w1s,
         me["W1"], vec(me, "b1"), me["W2"], vec(me, "b2"),
         vec(me, "g"), vec(me, "bt"), w1d],
        [(D, jnp.float32)] * 4, n_blocked=2)

    NCH_A = 64
    NCH_B = _NCHUNK - NCH_A
    EA = _NW * NCH_A * _CH
    idx0 = edge_index[0]
    idx1 = edge_index[1]
    i0a = idx0[:EA].reshape(_NW, NCH_A, _CH)
    i1a = idx1[:EA].reshape(_NW, NCH_A, _CH)
    i0b = idx0[EA:].reshape(_NW, NCH_B, _CH)
    i1b = idx1[EA:].reshape(_NW, NCH_B, _CH)

    sump_a = _sc_gather(gs, md, i0a, i1a, NCH_A)
    sump_b = _sc_gather(gs, md, i0b, i1b, NCH_B)

    ee = p["edge_emb"]
    ie = p["in_edge"]
    RE = 2560
    OFF = EA // RE
    eft = grid2mesh_efeat.T
    ew = [ee["W1"], vec(ee, "b1"), ee["W2"], vec(ee, "b2"),
          vec(ee, "g"), vec(ee, "bt"),
          w1e, vec(ie, "b1"), ie["W2"], vec(ie, "b2"),
          vec(ie, "g"), vec(ie, "bt")]
    eshape = (jax.ShapeDtypeStruct((E, D), jnp.float32),
              jax.ShapeDtypeStruct((E, D), jnp.float32))

    eins_a = [eft, sump_a] + ew
    e_out_a, e_new_a = pl.pallas_call(
        _edge_body,
        grid=(OFF,),
        in_specs=[pl.BlockSpec((4, RE), lambda i: (0, i)),
                  _row_spec(RE, D)] + [_full_spec(a) for a in ew],
        out_specs=(_row_spec(RE, D), _row_spec(RE, D)),
        out_shape=eshape,
    )(*eins_a)

    zeros = jnp.zeros((_STRIPE, D), jnp.float32)
    partials_a = _sc_scatter(e_new_a, i1a, zeros, 0, NCH_A)

    def _edge_body_b(eft, sump, *args):
        _edge_body(eft, sump, *args[:12], args[14], args[15])

    eins_b = [eft, sump_b] + ew + [e_out_a, e_new_a]
    e_out, e_new = pl.pallas_call(
        _edge_body_b,
        grid=(E // RE - OFF,),
        in_specs=[pl.BlockSpec((4, RE), lambda i: (0, i + OFF)),
                  _row_spec(RE, D)] + [_full_spec(a) for a in ew]
                 + [pl.BlockSpec(memory_space=pl.ANY),
                    pl.BlockSpec(memory_space=pl.ANY)],
        out_specs=(pl.BlockSpec((RE, D), lambda i: (i + OFF, 0)),
                   pl.BlockSpec((RE, D), lambda i: (i + OFF, 0))),
        out_shape=eshape,
        input_output_aliases={len(eins_b) - 2: 0, len(eins_b) - 1: 1},
    )(*eins_b)

    partials_b = _sc_scatter(e_new, i1b, zeros, EA, NCH_B)

    inn = p["in_node"]
    nins = [m, partials_a, partials_a, partials_b, partials_b,
            wa, wm, vec(inn, "b1"), inn["W2"],
            vec(inn, "b2"), vec(inn, "g"), vec(inn, "bt")]
    m_out = pl.pallas_call(
        _node_body,
        grid=(25,),
        in_specs=[_row_spec(400, D),
                  pl.BlockSpec((1, 400, D), lambda i: (0, i, 0)),
                  pl.BlockSpec((1, 400, D), lambda i: (1, i, 0)),
                  pl.BlockSpec((1, 400, D), lambda i: (0, i, 0)),
                  pl.BlockSpec((1, 400, D), lambda i: (1, i, 0))]
                 + [_full_spec(a) for a in nins[5:]],
        out_specs=_row_spec(400, D),
        out_shape=jax.ShapeDtypeStruct((N_MESH, D), jnp.float32),
    )(*nins)

    return (g_out, m_out, e_out)


# speedup vs baseline: 6.2434x; 1.1035x over previous
import functools

import jax
import jax.numpy as jnp
from jax import lax
from jax.experimental import pallas as pl
from jax.experimental.pallas import tpu as pltpu
from jax.experimental.pallas import tpu_sc as plsc

D = 128
N_GRID = 10000
N_MESH = 10000
E = 320000

_NC = 2
_NS = 16
_NW = _NC * _NS
_EPW = E // _NW
_CH = 80
_NCHUNK = _EPW // _CH
_NPAD = 10240
_STRIPE = _NPAD // _NS


def _silu(x):
    return x * jax.nn.sigmoid(x)


def _ln(y, g, bt):
    mu = jnp.mean(y, axis=-1, keepdims=True)
    yc = y - mu
    var = jnp.mean(yc * yc, axis=-1, keepdims=True)
    return yc * lax.rsqrt(var + 1e-5) * g + bt


def _pack_ilv(x):
    xb = x.astype(jnp.bfloat16)
    return jnp.stack([xb[:, :64], xb[:, 64:]], axis=-1).reshape(x.shape)


def _full_spec(a):
    nd = a.ndim
    return pl.BlockSpec(a.shape, lambda i, _n=nd: (0,) * _n)


def _row_spec(rows, cols):
    return pl.BlockSpec((rows, cols), lambda i: (i, 0))


def _gm_body(xg, xm,
             gw1, gb1, gw2, gb2, gg, gbt,
             nw1, nb1, nw2, nb2, ng, nbt, ws,
             mw1, mb1, mw2, mb2, mg, mbt, wd,
             gout_ref, gs_ref, m_ref, md_ref):
    h = _silu(jnp.dot(xg[...], gw1[...], preferred_element_type=jnp.float32)
              + gb1[...])
    g = _ln(jnp.dot(h, gw2[...], preferred_element_type=jnp.float32)
            + gb2[...], gg[...], gbt[...])
    h2 = _silu(jnp.dot(g, nw1[...], preferred_element_type=jnp.float32)
               + nb1[...])
    y2 = jnp.dot(h2, nw2[...], preferred_element_type=jnp.float32) + nb2[...]
    gout_ref[...] = g + _ln(y2, ng[...], nbt[...])
    gs_ref[...] = jnp.dot(g, ws[...], preferred_element_type=jnp.float32)
    hm = _silu(jnp.dot(xm[...], mw1[...], preferred_element_type=jnp.float32)
               + mb1[...])
    m = _ln(jnp.dot(hm, mw2[...], preferred_element_type=jnp.float32)
            + mb2[...], mg[...], mbt[...])
    m_ref[...] = m
    md_ref[...] = jnp.dot(m, wd[...], preferred_element_type=jnp.float32)


def _edge_body(eft, sump, ew1, eb1, ew2, eb2, eg, ebt,
               we, ib1, iw2, ib2, ig, ibt, eout_ref, enew_ref):
    h0pre = jax.lax.dot_general(
        eft[...], ew1[...], (((0,), (0,)), ((), ())),
        preferred_element_type=jnp.float32)
    h0 = _silu(h0pre + eb1[...])
    e = _ln(jnp.dot(h0, ew2[...], preferred_element_type=jnp.float32)
            + eb2[...], eg[...], ebt[...])
    pre = (jnp.dot(e, we[...], preferred_element_type=jnp.float32) + ib1[...]
           + sump[...])
    h = _silu(pre)
    en = _ln(jnp.dot(h, iw2[...], preferred_element_type=jnp.float32)
             + ib2[...], ig[...], ibt[...])
    enew_ref[...] = en
    eout_ref[...] = e + en


def _node_body(m, pa0, pa1, pb0, pb1, wa, wm, b1, w2, b2, gg, bt, mout_ref):
    agg = pa0[0] + pa1[0] + pb0[0] + pb1[0]
    h = _silu(jnp.dot(agg, wa[...], preferred_element_type=jnp.float32)
              + jnp.dot(m[...], wm[...], preferred_element_type=jnp.float32)
              + b1[...])
    mn = _ln(jnp.dot(h, w2[...], preferred_element_type=jnp.float32) + b2[...],
             gg[...], bt[...])
    mout_ref[...] = m[...] + mn


def _run_rows(body, grid_n, row_block, ins, outs, n_blocked=1):
    out_shape = tuple(jax.ShapeDtypeStruct((grid_n * row_block, c), dt)
                      for c, dt in outs)
    in_specs = [_row_spec(row_block, a.shape[-1]) if k < n_blocked
                else _full_spec(a) for k, a in enumerate(ins)]
    out_specs = tuple(_row_spec(row_block, c) for c, _ in outs)
    one = len(outs) == 1
    return pl.pallas_call(
        body,
        grid=(grid_n,),
        in_specs=in_specs,
        out_specs=out_specs[0] if one else out_specs,
        out_shape=out_shape[0] if one else out_shape,
    )(*ins)


@functools.lru_cache(maxsize=None)
def _build_sc_gather(nch):
    mesh = plsc.VectorSubcoreMesh(core_axis_name="c", subcore_axis_name="s")
    P = 4
    NRING = (nch // P) * P
    epw = nch * _CH

    @functools.partial(
        pl.kernel, mesh=mesh,
        out_type=jax.ShapeDtypeStruct((_NW * epw, D), jnp.float32),
        scratch_types=[pltpu.VMEM((nch, _CH), jnp.int32),
                       pltpu.VMEM((nch, _CH), jnp.int32)]
                      + [pltpu.VMEM((_CH, D), jnp.float32)] * (2 * P)
                      + [pltpu.SemaphoreType.DMA] * (3 * P),
    )
    def sc_gather(gs_hbm, md_hbm, idx0_hbm, idx1_hbm, sum_hbm,
                  idx0_v, idx1_v, *bufsems):
        ra = bufsems[0:P]
        rb = bufsems[P:2 * P]
        sga = bufsems[2 * P:3 * P]
        sgb = bufsems[3 * P:4 * P]
        sw = bufsems[4 * P:5 * P]
        wid = lax.axis_index("s") * _NC + lax.axis_index("c")
        pltpu.sync_copy(idx0_hbm.at[wid], idx0_v)
        pltpu.sync_copy(idx1_hbm.at[wid], idx1_v)

        def add_into(dst, src):
            def add_body(i, carry):
                for q in range(D // 16):
                    o = q * 16
                    dst[i, pl.ds(o, 16)] = (dst[i, pl.ds(o, 16)]
                                            + src[i, pl.ds(o, 16)])
                return carry
            lax.fori_loop(0, _CH, add_body, 0)

        def start(c, u):
            pltpu.async_copy(gs_hbm.at[idx0_v.at[c]], ra[u], sga[u])
            pltpu.async_copy(md_hbm.at[idx1_v.at[c]], rb[u], sgb[u])

        def wait_gather(c, u):
            pltpu.make_async_copy(gs_hbm.at[idx0_v.at[c]], ra[u],
                                  sga[u]).wait()
            pltpu.make_async_copy(md_hbm.at[idx1_v.at[c]], rb[u],
                                  sgb[u]).wait()

        for u in range(P):
            start(u, u)

        def body(k, carry):
            for u in range(P):
                c = P * k + u
                b = wid * epw + c * _CH
                wait_gather(c, u)
                add_into(ra[u], rb[u])
                pltpu.async_copy(ra[u], sum_hbm.at[pl.ds(b, _CH)], sw[u])
            for u in range(P):
                c = P * k + u
                cn = c + P
                b = wid * epw + c * _CH
                pltpu.make_async_copy(ra[u], sum_hbm.at[pl.ds(b, _CH)],
                                      sw[u]).wait()

                @pl.when(cn < NRING)
                def _():
                    start(cn, u)
            return carry

        lax.fori_loop(0, NRING // P, body, 0)
        for ct in range(NRING, nch):
            bt = wid * epw + ct * _CH
            start(ct, 0)
            wait_gather(ct, 0)
            add_into(ra[0], rb[0])
            pltpu.sync_copy(ra[0], sum_hbm.at[pl.ds(bt, _CH)])

    return sc_gather


@functools.lru_cache(maxsize=None)
def _build_sc_scatter(e0, nch):
    mesh = plsc.VectorSubcoreMesh(core_axis_name="c", subcore_axis_name="s")

    P = 3
    NRING = (nch // P) * P
    epw = nch * _CH

    @functools.partial(
        pl.kernel, mesh=mesh,
        out_type=jax.ShapeDtypeStruct((_NC, _NPAD, D), jnp.float32),
        scratch_types=[pltpu.VMEM((1, _CH), jnp.int32)] * P
                      + [pltpu.VMEM((_CH, D), jnp.float32)] * P
                      + [pltpu.VMEM_SHARED((_NPAD, D), jnp.float32)]
                      + [pltpu.SemaphoreType.DMA] * (3 * P),
    )
    def sc_scatter(enew_hbm, idx1_hbm, zeros_hbm, out_hbm, *rest):
        ibuf = rest[0:P]
        rbuf = rest[P:2 * P]
        acc_sh = rest[2 * P]
        sr = rest[2 * P + 1:3 * P + 1]
        si = rest[3 * P + 1:4 * P + 1]
        sa = rest[4 * P + 1:5 * P + 1]
        cid = lax.axis_index("c")
        sid = lax.axis_index("s")
        wid = sid * _NC + cid
        pltpu.sync_copy(zeros_hbm, acc_sh.at[pl.ds(sid * _STRIPE, _STRIPE)])
        plsc.subcore_barrier()

        def start_read(c, u):
            b = e0 + wid * epw + c * _CH
            pltpu.async_copy(enew_hbm.at[pl.ds(b, _CH)], rbuf[u], sr[u])

        def start_idx(c, u):
            pltpu.async_copy(idx1_hbm.at[wid].at[pl.ds(c, 1)], ibuf[u],
                             si[u])

        def wait_read(c, u):
            b = e0 + wid * epw + c * _CH
            pltpu.make_async_copy(enew_hbm.at[pl.ds(b, _CH)], rbuf[u],
                                  sr[u]).wait()
            pltpu.make_async_copy(idx1_hbm.at[wid].at[pl.ds(c, 1)], ibuf[u],
                                  si[u]).wait()

        for u in range(P):
            start_read(u, u)
            start_idx(u, u)

        def body(k, carry):
            for u in range(P):
                c = P * k + u
                wait_read(c, u)
                pltpu.async_copy(rbuf[u], acc_sh.at[ibuf[u].at[0]], sa[u],
                                 add=True)
            for u in range(P):
                c = P * k + u
                pltpu.make_async_copy(rbuf[u], acc_sh.at[ibuf[u].at[0]],
                                      sa[u]).wait()

                @pl.when(c + P < NRING)
                def _():
                    start_read(c + P, u)
                    start_idx(c + P, u)
            return carry

        lax.fori_loop(0, NRING // P, body, 0)
        for ct in range(NRING, nch):
            start_read(ct, 0)
            start_idx(ct, 0)
            wait_read(ct, 0)
            pltpu.sync_copy(rbuf[0], acc_sh.at[ibuf[0].at[0]], add=True)
        plsc.subcore_barrier()
        pltpu.sync_copy(acc_sh.at[pl.ds(sid * _STRIPE, _STRIPE)],
                        out_hbm.at[cid, pl.ds(sid * _STRIPE, _STRIPE)])

    return sc_scatter


def _sc_gather(gs, md, idx0, idx1, nch):
    return _build_sc_gather(nch)(gs, md, idx0, idx1)


def _sc_scatter(e_new, idx1, zeros, e0, nch):
    return _build_sc_scatter(e0, nch)(e_new, idx1, zeros)


def kernel(grid_nfeat, mesh_nfeat, edge_index, grid2mesh_efeat, params):
    p = params

    def vec(w, name):
        return w[name].reshape(1, -1)

    in_w1 = p["in_edge"]["W1"]
    w1e, w1s, w1d = in_w1[0:D], in_w1[D:2 * D], in_w1[2 * D:3 * D]
    in_node_w1 = p["in_node"]["W1"]
    wa, wm = in_node_w1[0:D], in_node_w1[D:2 * D]

    ge = p["grid_emb"]
    gn = p["grid_node"]
    me = p["mesh_emb"]
    g_out, gs, m, md = _run_rows(
        _gm_body, 25, 400,
        [grid_nfeat, mesh_nfeat,
         ge["W1"], vec(ge, "b1"), ge["W2"], vec(ge, "b2"),
         vec(ge, "g"), vec(ge, "bt"),
         gn["W1"], vec(gn, "b1"), gn["W2"], vec(gn, "b2"),
         vec(gn, "g"), vec(gn, "bt"), w1s,
         me["W1"], vec(me, "b1"), me["W2"], vec(me, "b2"),
         vec(me, "g"), vec(me, "bt"), w1d],
        [(D, jnp.float32)] * 4, n_blocked=2)

    NCH_A = 64
    NCH_B = _NCHUNK - NCH_A
    EA = _NW * NCH_A * _CH
    idx0 = edge_index[0]
    idx1 = edge_index[1]
    i0a = idx0[:EA].reshape(_NW, NCH_A, _CH)
    i1a = idx1[:EA].reshape(_NW, NCH_A, _CH)
    i0b = idx0[EA:].reshape(_NW, NCH_B, _CH)
    i1b = idx1[EA:].reshape(_NW, NCH_B, _CH)

    sump_a = _sc_gather(gs, md, i0a, i1a, NCH_A)
    sump_b = _sc_gather(gs, md, i0b, i1b, NCH_B)

    ee = p["edge_emb"]
    ie = p["in_edge"]
    RE = 2560
    OFF = EA // RE
    eft = grid2mesh_efeat.T
    ew = [ee["W1"], vec(ee, "b1"), ee["W2"], vec(ee, "b2"),
          vec(ee, "g"), vec(ee, "bt"),
          w1e, vec(ie, "b1"), ie["W2"], vec(ie, "b2"),
          vec(ie, "g"), vec(ie, "bt")]
    eshape = (jax.ShapeDtypeStruct((E, D), jnp.float32),
              jax.ShapeDtypeStruct((E, D), jnp.float32))

    eins_a = [eft, sump_a] + ew
    e_out_a, e_new_a = pl.pallas_call(
        _edge_body,
        grid=(OFF,),
        in_specs=[pl.BlockSpec((4, RE), lambda i: (0, i)),
                  _row_spec(RE, D)] + [_full_spec(a) for a in ew],
        out_specs=(_row_spec(RE, D), _row_spec(RE, D)),
        out_shape=(jax.ShapeDtypeStruct((E, D), jnp.float32),
                   jax.ShapeDtypeStruct((EA, D), jnp.float32)),
    )(*eins_a)

    zeros = jnp.zeros((_STRIPE, D), jnp.float32)
    partials_a = _sc_scatter(e_new_a, i1a, zeros, 0, NCH_A)

    def _edge_body_b(eft, sump, *args):
        _edge_body(eft, sump, *args[:12], args[13], args[14])

    eins_b = [eft, sump_b] + ew + [e_out_a]
    e_out, e_new_b = pl.pallas_call(
        _edge_body_b,
        grid=(E // RE - OFF,),
        in_specs=[pl.BlockSpec((4, RE), lambda i: (0, i + OFF)),
                  _row_spec(RE, D)] + [_full_spec(a) for a in ew]
                 + [pl.BlockSpec(memory_space=pl.ANY)],
        out_specs=(pl.BlockSpec((RE, D), lambda i: (i + OFF, 0)),
                   _row_spec(RE, D)),
        out_shape=(jax.ShapeDtypeStruct((E, D), jnp.float32),
                   jax.ShapeDtypeStruct((E - EA, D), jnp.float32)),
        input_output_aliases={len(eins_b) - 1: 0},
    )(*eins_b)

    partials_b = _sc_scatter(e_new_b, i1b, zeros, 0, NCH_B)

    inn = p["in_node"]
    nins = [m, partials_a, partials_a, partials_b, partials_b,
            wa, wm, vec(inn, "b1"), inn["W2"],
            vec(inn, "b2"), vec(inn, "g"), vec(inn, "bt")]
    m_out = pl.pallas_call(
        _node_body,
        grid=(25,),
        in_specs=[_row_spec(400, D),
                  pl.BlockSpec((1, 400, D), lambda i: (0, i, 0)),
                  pl.BlockSpec((1, 400, D), lambda i: (1, i, 0)),
                  pl.BlockSpec((1, 400, D), lambda i: (0, i, 0)),
                  pl.BlockSpec((1, 400, D), lambda i: (1, i, 0))]
                 + [_full_spec(a) for a in nins[5:]],
        out_specs=_row_spec(400, D),
        out_shape=jax.ShapeDtypeStruct((N_MESH, D), jnp.float32),
    )(*nins)

    return (g_out, m_out, e_out)


# speedup vs baseline: 6.2692x; 1.0041x over previous
import functools

import jax
import jax.numpy as jnp
from jax import lax
from jax.experimental import pallas as pl
from jax.experimental.pallas import tpu as pltpu
from jax.experimental.pallas import tpu_sc as plsc

D = 128
N_GRID = 10000
N_MESH = 10000
E = 320000

_NC = 2
_NS = 16
_NW = _NC * _NS
_EPW = E // _NW
_CH = 80
_NCHUNK = _EPW // _CH
_NPAD = 10240
_STRIPE = _NPAD // _NS


def _silu(x):
    return x * jax.nn.sigmoid(x)


def _ln(y, g, bt):
    mu = jnp.mean(y, axis=-1, keepdims=True)
    yc = y - mu
    var = jnp.mean(yc * yc, axis=-1, keepdims=True)
    return yc * lax.rsqrt(var + 1e-5) * g + bt


def _pack_ilv(x):
    xb = x.astype(jnp.bfloat16)
    return jnp.stack([xb[:, :64], xb[:, 64:]], axis=-1).reshape(x.shape)


def _full_spec(a):
    nd = a.ndim
    return pl.BlockSpec(a.shape, lambda i, _n=nd: (0,) * _n)


def _row_spec(rows, cols):
    return pl.BlockSpec((rows, cols), lambda i: (i, 0))


def _gm_body(xg, xm,
             gw1, gb1, gw2, gb2, gg, gbt, ws,
             mw1, mb1, mw2, mb2, mg, mbt, wd,
             g_ref, gs_ref, m_ref, md_ref):
    h = _silu(jnp.dot(xg[...], gw1[...], preferred_element_type=jnp.float32)
              + gb1[...])
    g = _ln(jnp.dot(h, gw2[...], preferred_element_type=jnp.float32)
            + gb2[...], gg[...], gbt[...])
    g_ref[...] = g
    gs_ref[...] = jnp.dot(g, ws[...], preferred_element_type=jnp.float32)
    hm = _silu(jnp.dot(xm[...], mw1[...], preferred_element_type=jnp.float32)
               + mb1[...])
    m = _ln(jnp.dot(hm, mw2[...], preferred_element_type=jnp.float32)
            + mb2[...], mg[...], mbt[...])
    m_ref[...] = m
    md_ref[...] = jnp.dot(m, wd[...], preferred_element_type=jnp.float32)


def _gout_body(g, nw1, nb1, nw2, nb2, ng, nbt, gout_ref):
    h2 = _silu(jnp.dot(g[...], nw1[...], preferred_element_type=jnp.float32)
               + nb1[...])
    y2 = jnp.dot(h2, nw2[...], preferred_element_type=jnp.float32) + nb2[...]
    gout_ref[...] = g[...] + _ln(y2, ng[...], nbt[...])


def _edge_body(eft, sump, ew1, eb1, ew2, eb2, eg, ebt,
               we, ib1, iw2, ib2, ig, ibt, eout_ref, enew_ref):
    h0pre = jax.lax.dot_general(
        eft[...], ew1[...], (((0,), (0,)), ((), ())),
        preferred_element_type=jnp.float32)
    h0 = _silu(h0pre + eb1[...])
    e = _ln(jnp.dot(h0, ew2[...], preferred_element_type=jnp.float32)
            + eb2[...], eg[...], ebt[...])
    pre = (jnp.dot(e, we[...], preferred_element_type=jnp.float32) + ib1[...]
           + sump[...])
    h = _silu(pre)
    en = _ln(jnp.dot(h, iw2[...], preferred_element_type=jnp.float32)
             + ib2[...], ig[...], ibt[...])
    enew_ref[...] = en
    eout_ref[...] = e + en


def _node_body(m, pa0, pa1, pb0, pb1, wa, wm, b1, w2, b2, gg, bt, mout_ref):
    agg = pa0[0] + pa1[0] + pb0[0] + pb1[0]
    h = _silu(jnp.dot(agg, wa[...], preferred_element_type=jnp.float32)
              + jnp.dot(m[...], wm[...], preferred_element_type=jnp.float32)
              + b1[...])
    mn = _ln(jnp.dot(h, w2[...], preferred_element_type=jnp.float32) + b2[...],
             gg[...], bt[...])
    mout_ref[...] = m[...] + mn


def _run_rows(body, grid_n, row_block, ins, outs, n_blocked=1):
    out_shape = tuple(jax.ShapeDtypeStruct((grid_n * row_block, c), dt)
                      for c, dt in outs)
    in_specs = [_row_spec(row_block, a.shape[-1]) if k < n_blocked
                else _full_spec(a) for k, a in enumerate(ins)]
    out_specs = tuple(_row_spec(row_block, c) for c, _ in outs)
    one = len(outs) == 1
    return pl.pallas_call(
        body,
        grid=(grid_n,),
        in_specs=in_specs,
        out_specs=out_specs[0] if one else out_specs,
        out_shape=out_shape[0] if one else out_shape,
    )(*ins)


@functools.lru_cache(maxsize=None)
def _build_sc_gather(nch):
    mesh = plsc.VectorSubcoreMesh(core_axis_name="c", subcore_axis_name="s")
    P = 4
    NRING = (nch // P) * P
    epw = nch * _CH

    @functools.partial(
        pl.kernel, mesh=mesh,
        out_type=jax.ShapeDtypeStruct((_NW * epw, D), jnp.float32),
        scratch_types=[pltpu.VMEM((nch, _CH), jnp.int32),
                       pltpu.VMEM((nch, _CH), jnp.int32)]
                      + [pltpu.VMEM((_CH, D), jnp.float32)] * (2 * P)
                      + [pltpu.SemaphoreType.DMA] * (3 * P),
    )
    def sc_gather(gs_hbm, md_hbm, idx0_hbm, idx1_hbm, sum_hbm,
                  idx0_v, idx1_v, *bufsems):
        ra = bufsems[0:P]
        rb = bufsems[P:2 * P]
        sga = bufsems[2 * P:3 * P]
        sgb = bufsems[3 * P:4 * P]
        sw = bufsems[4 * P:5 * P]
        wid = lax.axis_index("s") * _NC + lax.axis_index("c")
        pltpu.sync_copy(idx0_hbm.at[wid], idx0_v)
        pltpu.sync_copy(idx1_hbm.at[wid], idx1_v)

        def add_into(dst, src):
            def add_body(i, carry):
                for q in range(D // 16):
                    o = q * 16
                    dst[i, pl.ds(o, 16)] = (dst[i, pl.ds(o, 16)]
                                            + src[i, pl.ds(o, 16)])
                return carry
            lax.fori_loop(0, _CH, add_body, 0)

        def start(c, u):
            pltpu.async_copy(gs_hbm.at[idx0_v.at[c]], ra[u], sga[u])
            pltpu.async_copy(md_hbm.at[idx1_v.at[c]], rb[u], sgb[u])

        def wait_gather(c, u):
            pltpu.make_async_copy(gs_hbm.at[idx0_v.at[c]], ra[u],
                                  sga[u]).wait()
            pltpu.make_async_copy(md_hbm.at[idx1_v.at[c]], rb[u],
                                  sgb[u]).wait()

        for u in range(P):
            start(u, u)

        def body(k, carry):
            for u in range(P):
                c = P * k + u
                b = wid * epw + c * _CH
                wait_gather(c, u)
                add_into(ra[u], rb[u])
                pltpu.async_copy(ra[u], sum_hbm.at[pl.ds(b, _CH)], sw[u])
            for u in range(P):
                c = P * k + u
                cn = c + P
                b = wid * epw + c * _CH
                pltpu.make_async_copy(ra[u], sum_hbm.at[pl.ds(b, _CH)],
                                      sw[u]).wait()

                @pl.when(cn < NRING)
                def _():
                    start(cn, u)
            return carry

        lax.fori_loop(0, NRING // P, body, 0)
        for ct in range(NRING, nch):
            bt = wid * epw + ct * _CH
            start(ct, 0)
            wait_gather(ct, 0)
            add_into(ra[0], rb[0])
            pltpu.sync_copy(ra[0], sum_hbm.at[pl.ds(bt, _CH)])

    return sc_gather


@functools.lru_cache(maxsize=None)
def _build_sc_scatter(e0, nch):
    mesh = plsc.VectorSubcoreMesh(core_axis_name="c", subcore_axis_name="s")

    P = 3
    NRING = (nch // P) * P
    epw = nch * _CH

    @functools.partial(
        pl.kernel, mesh=mesh,
        out_type=jax.ShapeDtypeStruct((_NC, _NPAD, D), jnp.float32),
        scratch_types=[pltpu.VMEM((1, _CH), jnp.int32)] * P
                      + [pltpu.VMEM((_CH, D), jnp.float32)] * P
                      + [pltpu.VMEM_SHARED((_NPAD, D), jnp.float32)]
                      + [pltpu.SemaphoreType.DMA] * (3 * P),
    )
    def sc_scatter(enew_hbm, idx1_hbm, zeros_hbm, out_hbm, *rest):
        ibuf = rest[0:P]
        rbuf = rest[P:2 * P]
        acc_sh = rest[2 * P]
        sr = rest[2 * P + 1:3 * P + 1]
        si = rest[3 * P + 1:4 * P + 1]
        sa = rest[4 * P + 1:5 * P + 1]
        cid = lax.axis_index("c")
        sid = lax.axis_index("s")
        wid = sid * _NC + cid
        pltpu.sync_copy(zeros_hbm, acc_sh.at[pl.ds(sid * _STRIPE, _STRIPE)])
        plsc.subcore_barrier()

        def start_read(c, u):
            b = e0 + wid * epw + c * _CH
            pltpu.async_copy(enew_hbm.at[pl.ds(b, _CH)], rbuf[u], sr[u])

        def start_idx(c, u):
            pltpu.async_copy(idx1_hbm.at[wid].at[pl.ds(c, 1)], ibuf[u],
                             si[u])

        def wait_read(c, u):
            b = e0 + wid * epw + c * _CH
            pltpu.make_async_copy(enew_hbm.at[pl.ds(b, _CH)], rbuf[u],
                                  sr[u]).wait()
            pltpu.make_async_copy(idx1_hbm.at[wid].at[pl.ds(c, 1)], ibuf[u],
                                  si[u]).wait()

        for u in range(P):
            start_read(u, u)
            start_idx(u, u)

        def body(k, carry):
            for u in range(P):
                c = P * k + u
                wait_read(c, u)
                pltpu.async_copy(rbuf[u], acc_sh.at[ibuf[u].at[0]], sa[u],
                                 add=True)
            for u in range(P):
                c = P * k + u
                pltpu.make_async_copy(rbuf[u], acc_sh.at[ibuf[u].at[0]],
                                      sa[u]).wait()

                @pl.when(c + P < NRING)
                def _():
                    start_read(c + P, u)
                    start_idx(c + P, u)
            return carry

        lax.fori_loop(0, NRING // P, body, 0)
        for ct in range(NRING, nch):
            start_read(ct, 0)
            start_idx(ct, 0)
            wait_read(ct, 0)
            pltpu.sync_copy(rbuf[0], acc_sh.at[ibuf[0].at[0]], add=True)
        plsc.subcore_barrier()
        pltpu.sync_copy(acc_sh.at[pl.ds(sid * _STRIPE, _STRIPE)],
                        out_hbm.at[cid, pl.ds(sid * _STRIPE, _STRIPE)])

    return sc_scatter


def _sc_gather(gs, md, idx0, idx1, nch):
    return _build_sc_gather(nch)(gs, md, idx0, idx1)


def _sc_scatter(e_new, idx1, zeros, e0, nch):
    return _build_sc_scatter(e0, nch)(e_new, idx1, zeros)


def kernel(grid_nfeat, mesh_nfeat, edge_index, grid2mesh_efeat, params):
    p = params

    def vec(w, name):
        return w[name].reshape(1, -1)

    in_w1 = p["in_edge"]["W1"]
    w1e, w1s, w1d = in_w1[0:D], in_w1[D:2 * D], in_w1[2 * D:3 * D]
    in_node_w1 = p["in_node"]["W1"]
    wa, wm = in_node_w1[0:D], in_node_w1[D:2 * D]

    ge = p["grid_emb"]
    gn = p["grid_node"]
    me = p["mesh_emb"]
    g, gs, m, md = _run_rows(
        _gm_body, 25, 400,
        [grid_nfeat, mesh_nfeat,
         ge["W1"], vec(ge, "b1"), ge["W2"], vec(ge, "b2"),
         vec(ge, "g"), vec(ge, "bt"), w1s,
         me["W1"], vec(me, "b1"), me["W2"], vec(me, "b2"),
         vec(me, "g"), vec(me, "bt"), w1d],
        [(D, jnp.float32)] * 4, n_blocked=2)

    g_out = _run_rows(
        _gout_body, 25, 400,
        [g, gn["W1"], vec(gn, "b1"), gn["W2"], vec(gn, "b2"),
         vec(gn, "g"), vec(gn, "bt")],
        [(D, jnp.float32)])

    NCH_A = 57
    NCH_B = _NCHUNK - NCH_A
    EA = _NW * NCH_A * _CH
    idx0 = edge_index[0]
    idx1 = edge_index[1]
    i0a = idx0[:EA].reshape(_NW, NCH_A, _CH)
    i1a = idx1[:EA].reshape(_NW, NCH_A, _CH)
    i0b = idx0[EA:].reshape(_NW, NCH_B, _CH)
    i1b = idx1[EA:].reshape(_NW, NCH_B, _CH)

    sump_a = _sc_gather(gs, md, i0a, i1a, NCH_A)
    sump_b = _sc_gather(gs, md, i0b, i1b, NCH_B)

    ee = p["edge_emb"]
    ie = p["in_edge"]
    RE = 2560
    OFF = EA // RE
    eft = grid2mesh_efeat.T
    ew = [ee["W1"], vec(ee, "b1"), ee["W2"], vec(ee, "b2"),
          vec(ee, "g"), vec(ee, "bt"),
          w1e, vec(ie, "b1"), ie["W2"], vec(ie, "b2"),
          vec(ie, "g"), vec(ie, "bt")]
    eshape = (jax.ShapeDtypeStruct((E, D), jnp.float32),
              jax.ShapeDtypeStruct((E, D), jnp.float32))

    eins_a = [eft, sump_a] + ew
    e_out_a, e_new_a = pl.pallas_call(
        _edge_body,
        grid=(OFF,),
        in_specs=[pl.BlockSpec((4, RE), lambda i: (0, i)),
                  _row_spec(RE, D)] + [_full_spec(a) for a in ew],
        out_specs=(_row_spec(RE, D), _row_spec(RE, D)),
        out_shape=(jax.ShapeDtypeStruct((E, D), jnp.float32),
                   jax.ShapeDtypeStruct((EA, D), jnp.float32)),
    )(*eins_a)

    zeros = jnp.zeros((_STRIPE, D), jnp.float32)
    partials_a = _sc_scatter(e_new_a, i1a, zeros, 0, NCH_A)

    def _edge_body_b(eft, sump, *args):
        _edge_body(eft, sump, *args[:12], args[13], args[14])

    eins_b = [eft, sump_b] + ew + [e_out_a]
    e_out, e_new_b = pl.pallas_call(
        _edge_body_b,
        grid=(E // RE - OFF,),
        in_specs=[pl.BlockSpec((4, RE), lambda i: (0, i + OFF)),
                  _row_spec(RE, D)] + [_full_spec(a) for a in ew]
                 + [pl.BlockSpec(memory_space=pl.ANY)],
        out_specs=(pl.BlockSpec((RE, D), lambda i: (i + OFF, 0)),
                   _row_spec(RE, D)),
        out_shape=(jax.ShapeDtypeStruct((E, D), jnp.float32),
                   jax.ShapeDtypeStruct((E - EA, D), jnp.float32)),
        input_output_aliases={len(eins_b) - 1: 0},
    )(*eins_b)

    partials_b = _sc_scatter(e_new_b, i1b, zeros, 0, NCH_B)

    inn = p["in_node"]
    nins = [m, partials_a, partials_a, partials_b, partials_b,
            wa, wm, vec(inn, "b1"), inn["W2"],
            vec(inn, "b2"), vec(inn, "g"), vec(inn, "bt")]
    m_out = pl.pallas_call(
        _node_body,
        grid=(25,),
        in_specs=[_row_spec(400, D),
                  pl.BlockSpec((1, 400, D), lambda i: (0, i, 0)),
                  pl.BlockSpec((1, 400, D), lambda i: (1, i, 0)),
                  pl.BlockSpec((1, 400, D), lambda i: (0, i, 0)),
                  pl.BlockSpec((1, 400, D), lambda i: (1, i, 0))]
                 + [_full_spec(a) for a in nins[5:]],
        out_specs=_row_spec(400, D),
        out_shape=jax.ShapeDtypeStruct((N_MESH, D), jnp.float32),
    )(*nins)

    return (g_out, m_out, e_out)
